# baseline (speedup 1.0000x reference)
import jax
import jax.numpy as jnp
import numpy as np
from jax import lax
from jax.experimental import pallas as pl
from jax.experimental.pallas import tpu as pltpu
from jax.experimental.pallas import tpu_sc as plsc

BATCH = 8192
DIM = 1024
VOCAB = 100000
NUM_PAIRS = BATCH // 2
NC = 2
NS = 16
LANES = 16
NW = NC * NS
PT = NUM_PAIRS // NW
CH = 8
NCH = PT // CH
NSS = NCH // 2
NG = PT // LANES
NK = DIM // LANES
NMASK = BATCH // LANES
SIGMOID_SCALE = 10.0


_U32 = np.uint32


def _tf2x32(k1, k2, c1, c2):
    rot0 = (13, 15, 26, 6)
    rot1 = (17, 29, 16, 24)
    ks0 = _U32(k1)
    ks1 = _U32(k2)
    ks2 = _U32(ks0 ^ ks1 ^ _U32(0x1BD11BDA))
    x0 = c1.astype(_U32)
    x1 = c2.astype(_U32)
    with np.errstate(over="ignore"):
        x0 = x0 + ks0
        x1 = x1 + ks1

        def rounds(x0, x1, rots):
            for r in rots:
                x0 = (x0 + x1).astype(_U32)
                x1 = ((x1 << _U32(r)) | (x1 >> _U32(32 - r))).astype(_U32)
                x1 = x1 ^ x0
            return x0, x1

        for i, rots in enumerate((rot0, rot1, rot0, rot1, rot0)):
            x0, x1 = rounds(x0, x1, rots)
            ka, kb = ((ks1, ks2), (ks2, ks0), (ks0, ks1),
                      (ks1, ks2), (ks2, ks0))[i]
            x0 = (x0 + ka).astype(_U32)
            x1 = (x1 + kb + _U32(i + 1)).astype(_U32)
    return x0, x1


def _np_split(kpair, num):
    b1, b2 = _tf2x32(kpair[0], kpair[1], np.zeros(num, _U32),
                     np.arange(num, dtype=_U32))
    return np.stack([b1, b2], axis=1)


def _np_bits(kpair, size):
    b1, b2 = _tf2x32(kpair[0], kpair[1], np.zeros(size, _U32),
                     np.arange(size, dtype=_U32))
    return b1 ^ b2


def _pair_constants():
    kd = np.array([0, 42], _U32)
    ki, kj, ko, ke = _np_split(kd, 4)

    def bits2(kp):
        kk = _np_split(kp, 2)
        return _np_bits(kk[0], NUM_PAIRS), _np_bits(kk[1], NUM_PAIRS)

    hbi, lbi = bits2(ki)
    hbj, lbj = bits2(kj)
    hbo, lbo = bits2(ko)
    hbe, lbe = bits2(ke)
    span = _U32(VOCAB)
    with np.errstate(over="ignore"):
        mult = _U32(65536) % span
        mult = _U32(mult * mult) % span
        emb = (((hbe % span) * mult + (lbe % span)) % span).astype(np.int32)
    return np.concatenate([hbi, lbi, hbj, lbj, hbo, lbo,
                           emb.view(np.uint32)])


_CONSTS = _pair_constants()


def _norm_body(h_ref, e_ref, hn_ref, en_ref):
    h = h_ref[...]
    e = e_ref[...]
    hn_ref[...] = jnp.sqrt(jnp.sum(h * h, axis=-1))
    en_ref[...] = jnp.sqrt(jnp.sum(e * e, axis=-1))


def _row_norms(hidden, target):
    h3 = hidden.reshape(64, 128, DIM)
    e3 = target.reshape(64, 128, DIM)
    hn, en = pl.pallas_call(
        _norm_body,
        grid=(8,),
        in_specs=[
            pl.BlockSpec((8, 128, DIM), lambda i: (i, 0, 0)),
            pl.BlockSpec((8, 128, DIM), lambda i: (i, 0, 0)),
        ],
        out_specs=[
            pl.BlockSpec((8, 128), lambda i: (i, 0)),
            pl.BlockSpec((8, 128), lambda i: (i, 0)),
        ],
        out_shape=[
            jax.ShapeDtypeStruct((64, 128), jnp.float32),
            jax.ShapeDtypeStruct((64, 128), jnp.float32),
        ],
    )(h3, e3)
    return hn.reshape(BATCH), en.reshape(BATCH)


_GATHER_DNUMS = lax.GatherDimensionNumbers(
    offset_dims=(), collapsed_slice_dims=(0,), start_index_map=(0,))


def _lane_perm(v, idx):
    return lax.gather(v, idx.reshape(LANES, 1), _GATHER_DNUMS, (1,),
                      mode=lax.GatherScatterMode.PROMISE_IN_BOUNDS)


def _lane_allsum(v):
    lanes = lax.iota(jnp.int32, LANES)
    for k in (1, 2, 4, 8):
        v = v + _lane_perm(v, lanes ^ k)
    return v


def _rsqrt_newton(x):
    i = plsc.bitcast(x, jnp.int32)
    i = jnp.int32(0x5F3759DF) - lax.shift_right_arithmetic(i, 1)
    y = plsc.bitcast(i, jnp.float32)
    for _ in range(3):
        y = y * (1.5 - 0.5 * x * y * y)
    return y


def _sigmoid_sq(gap):
    s = 1.0 / (1.0 + jnp.exp(-SIGMOID_SCALE * gap))
    d = s - 0.5
    return d * d


def _randint_vec(hb, lb, span):
    mult = jnp.uint32(65536) % span
    mult = (mult * mult) % span
    off = ((hb % span) * mult + (lb % span)) % span
    return off.astype(jnp.int32)


def _gather_start(table, idx_slice, buf, sem):
    pltpu.make_async_copy(table.at[idx_slice], buf, sem).start()


def _gather_wait(table, idx_slice, buf, sem):
    pltpu.make_async_copy(table.at[idx_slice], buf, sem).wait()


def _sc_body(h_hbm, e_hbm, w_hbm, hn_hbm, en_hbm, cb_hbm,
             out_hbm,
             hn_v, en_v, vi_v, gi_v, gj_v, go_v, we_v, weu_v,
             hbi_v, lbi_v, hbj_v, lbj_v, hbo_v, lbo_v,
             ba0, bb0, bc0, bd0, ba1, bb1, bc1, bd1, psum_v,
             sa0, sb0, sc0, sd0, sa1, sb1, sc1, sd1):
    wid = lax.axis_index("s") * NC + lax.axis_index("c")
    base = wid * PT
    pltpu.sync_copy(hn_hbm, hn_v)
    pltpu.sync_copy(en_hbm, en_v)
    pltpu.sync_copy(cb_hbm.at[pl.ds(0 * NUM_PAIRS + base, PT)], hbi_v)
    pltpu.sync_copy(cb_hbm.at[pl.ds(1 * NUM_PAIRS + base, PT)], lbi_v)
    pltpu.sync_copy(cb_hbm.at[pl.ds(2 * NUM_PAIRS + base, PT)], hbj_v)
    pltpu.sync_copy(cb_hbm.at[pl.ds(3 * NUM_PAIRS + base, PT)], lbj_v)
    pltpu.sync_copy(cb_hbm.at[pl.ds(4 * NUM_PAIRS + base, PT)], hbo_v)
    pltpu.sync_copy(cb_hbm.at[pl.ds(5 * NUM_PAIRS + base, PT)], lbo_v)
    pltpu.sync_copy(cb_hbm.at[pl.ds(6 * NUM_PAIRS + base, PT)], weu_v)

    zeros = jnp.zeros((LANES,), jnp.float32)
    lanes = lax.iota(jnp.int32, LANES)
    last = jnp.full((LANES,), LANES - 1, jnp.int32)

    vi_v[pl.ds(0, LANES)] = jnp.zeros((LANES,), jnp.int32)

    def mask_body(i, off):
        sl = pl.ds(i * LANES, LANES)
        m = (hn_v[sl] > 1e-8) & (en_v[sl] > 1e-8)
        mi = m.astype(jnp.int32)
        cs = plsc.cumsum(mi)
        plsc.store_scatter(vi_v, [off + cs - 1], lanes + i * LANES, mask=m)
        return off + plsc.all_reduce_population_count(m)

    vb_vec = lax.fori_loop(0, NMASK, mask_body,
                           jnp.zeros((LANES,), jnp.int32), unroll=4)

    span_i = jnp.maximum(vb_vec, 1).astype(jnp.uint32)
    span_j = jnp.maximum(vb_vec - 1, 1).astype(jnp.uint32)

    def gen_body(g, carry):
        sl = pl.ds(g * LANES, LANES)
        ii = _randint_vec(hbi_v[sl], lbi_v[sl], span_i)
        jj = _randint_vec(hbj_v[sl], lbj_v[sl], span_j)
        jj = jj + (jj >= ii).astype(jnp.int32)
        oo = _randint_vec(hbo_v[sl], lbo_v[sl], span_i)
        gi_v[sl] = plsc.load_gather(vi_v, [ii])
        gj_v[sl] = plsc.load_gather(vi_v, [jj])
        go_v[sl] = plsc.load_gather(vi_v, [oo])
        we_v[sl] = plsc.bitcast(weu_v[sl], jnp.int32)
        return carry

    lax.fori_loop(0, NG, gen_body, 0)

    def gi_sl(c):
        return gi_v.at[pl.ds(c * CH, CH)]

    def gj_sl(c):
        return gj_v.at[pl.ds(c * CH, CH)]

    def go_sl(c):
        return go_v.at[pl.ds(c * CH, CH)]

    def we_sl(c):
        return we_v.at[pl.ds(c * CH, CH)]

    def geo_issue(c, ba, bb, bc, bd, sa, sb, sc, sd):
        _gather_start(h_hbm, gi_sl(c), ba, sa)
        _gather_start(h_hbm, gj_sl(c), bb, sb)
        _gather_start(e_hbm, gi_sl(c), bc, sc)
        _gather_start(e_hbm, gj_sl(c), bd, sd)

    def geo_wait(c, ba, bb, bc, bd, sa, sb, sc, sd):
        _gather_wait(h_hbm, gi_sl(c), ba, sa)
        _gather_wait(h_hbm, gj_sl(c), bb, sb)
        _gather_wait(e_hbm, gi_sl(c), bc, sc)
        _gather_wait(e_hbm, gj_sl(c), bd, sd)

    def geo_dots8(ba, bb, bc, bd, dot_h, dot_e, lane_base):
        for p in range(CH):
            def kbody(k, acc, _p=p):
                ah, ae = acc
                ko = k * LANES
                ah = ah + ba[_p, pl.ds(ko, LANES)] * bb[_p, pl.ds(ko, LANES)]
                ae = ae + bc[_p, pl.ds(ko, LANES)] * bd[_p, pl.ds(ko, LANES)]
                return ah, ae
            ah, ae = lax.fori_loop(0, NK, kbody, (zeros, zeros), unroll=16)
            sel = lanes == (lane_base + p)
            dot_h = jnp.where(sel, _lane_allsum(ah), dot_h)
            dot_e = jnp.where(sel, _lane_allsum(ae), dot_e)
        return dot_h, dot_e

    geo_issue(0, ba0, bb0, bc0, bd0, sa0, sb0, sc0, sd0)

    def geo_step(s, carry):
        s_abs, s_loss = carry
        c0 = 2 * s
        c1 = c0 + 1
        geo_issue(c1, ba1, bb1, bc1, bd1, sa1, sb1, sc1, sd1)
        geo_wait(c0, ba0, bb0, bc0, bd0, sa0, sb0, sc0, sd0)
        dot_h, dot_e = geo_dots8(ba0, bb0, bc0, bd0, zeros, zeros, 0)

        @pl.when(s < NSS - 1)
        def _():
            geo_issue(c0 + 2, ba0, bb0, bc0, bd0, sa0, sb0, sc0, sd0)

        geo_wait(c1, ba1, bb1, bc1, bd1, sa1, sb1, sc1, sd1)
        dot_h, dot_e = geo_dots8(ba1, bb1, bc1, bd1, dot_h, dot_e, CH)

        sl = pl.ds(s * LANES, LANES)
        gi_vec = gi_v[sl]
        gj_vec = gj_v[sl]
        hn_i = plsc.load_gather(hn_v, [gi_vec])
        hn_j = plsc.load_gather(hn_v, [gj_vec])
        en_i = plsc.load_gather(en_v, [gi_vec])
        en_j = plsc.load_gather(en_v, [gj_vec])
        sim_h = dot_h / jnp.maximum(hn_i * hn_j, 1e-8)
        sim_e = dot_e / jnp.maximum(en_i * en_j, 1e-8)
        gap = sim_h - sim_e
        return s_abs + jnp.abs(gap), s_loss + _sigmoid_sq(gap)

    g_abs, g_loss = lax.fori_loop(0, NSS, geo_step, (zeros, zeros))

    def anc_issue(c, ba, bb, bc, sa, sb, sc):
        _gather_start(h_hbm, go_sl(c), ba, sa)
        _gather_start(w_hbm, we_sl(c), bb, sb)
        _gather_start(e_hbm, go_sl(c), bc, sc)

    def anc_wait(c, ba, bb, bc, sa, sb, sc):
        _gather_wait(h_hbm, go_sl(c), ba, sa)
        _gather_wait(w_hbm, we_sl(c), bb, sb)
        _gather_wait(e_hbm, go_sl(c), bc, sc)

    def anc_dots8(ba, bb, bc, dot_hw, dot_ew, dot_ww, lane_base):
        for p in range(CH):
            def kbody(k, acc, _p=p):
                aw, ew, ww = acc
                ko = k * LANES
                wv = bb[_p, pl.ds(ko, LANES)]
                aw = aw + ba[_p, pl.ds(ko, LANES)] * wv
                ew = ew + bc[_p, pl.ds(ko, LANES)] * wv
                ww = ww + wv * wv
                return aw, ew, ww
            aw, ew, ww = lax.fori_loop(0, NK, kbody, (zeros, zeros, zeros),
                                       unroll=16)
            sel = lanes == (lane_base + p)
            dot_hw = jnp.where(sel, _lane_allsum(aw), dot_hw)
            dot_ew = jnp.where(sel, _lane_allsum(ew), dot_ew)
            dot_ww = jnp.where(sel, _lane_allsum(ww), dot_ww)
        return dot_hw, dot_ew, dot_ww

    anc_issue(0, ba0, bb0, bc0, sa0, sb0, sc0)

    def anc_step(s, carry):
        s_abs, s_loss = carry
        c0 = 2 * s
        c1 = c0 + 1
        anc_issue(c1, ba1, bb1, bc1, sa1, sb1, sc1)
        anc_wait(c0, ba0, bb0, bc0, sa0, sb0, sc0)
        dot_hw, dot_ew, dot_ww = anc_dots8(
            ba0, bb0, bc0, zeros, zeros, zeros, 0)

        @pl.when(s < NSS - 1)
        def _():
            anc_issue(c0 + 2, ba0, bb0, bc0, sa0, sb0, sc0)

        anc_wait(c1, ba1, bb1, bc1, sa1, sb1, sc1)
        dot_hw, dot_ew, dot_ww = anc_dots8(
            ba1, bb1, bc1, dot_hw, dot_ew, dot_ww, CH)

        sl = pl.ds(s * LANES, LANES)
        go_vec = go_v[sl]
        hn_o = plsc.load_gather(hn_v, [go_vec])
        en_o = plsc.load_gather(en_v, [go_vec])
        wn2 = jnp.maximum(dot_ww, 1e-30)
        wn = wn2 * _rsqrt_newton(wn2)
        sim_h = dot_hw / jnp.maximum(hn_o * wn, 1e-8)
        sim_e = dot_ew / jnp.maximum(en_o * wn, 1e-8)
        gap = sim_h - sim_e
        return s_abs + jnp.abs(gap), s_loss + _sigmoid_sq(gap)

    a_abs, a_loss = lax.fori_loop(0, NSS, anc_step, (zeros, zeros))

    psum_v[0, :] = g_abs
    psum_v[1, :] = g_loss
    psum_v[2, :] = a_abs
    psum_v[3, :] = a_loss
    pltpu.sync_copy(psum_v, out_hbm.at[wid])


def _sc_pair_loss(h, e, w, hn, en):
    mesh = plsc.VectorSubcoreMesh(core_axis_name="c", subcore_axis_name="s")
    f = pl.kernel(
        _sc_body,
        mesh=mesh,
        compiler_params=pltpu.CompilerParams(needs_layout_passes=False),
        out_type=jax.ShapeDtypeStruct((NW, 4, LANES), jnp.float32),
        scratch_types=[
            pltpu.VMEM((BATCH,), jnp.float32),
            pltpu.VMEM((BATCH,), jnp.float32),
            pltpu.VMEM((BATCH,), jnp.int32),
            pltpu.VMEM((PT,), jnp.int32),
            pltpu.VMEM((PT,), jnp.int32),
            pltpu.VMEM((PT,), jnp.int32),
            pltpu.VMEM((PT,), jnp.int32),
            pltpu.VMEM((PT,), jnp.uint32),
            pltpu.VMEM((PT,), jnp.uint32),
            pltpu.VMEM((PT,), jnp.uint32),
            pltpu.VMEM((PT,), jnp.uint32),
            pltpu.VMEM((PT,), jnp.uint32),
            pltpu.VMEM((PT,), jnp.uint32),
            pltpu.VMEM((PT,), jnp.uint32),
            pltpu.VMEM((CH, DIM), jnp.float32),
            pltpu.VMEM((CH, DIM), jnp.float32),
            pltpu.VMEM((CH, DIM), jnp.float32),
            pltpu.VMEM((CH, DIM), jnp.float32),
            pltpu.VMEM((CH, DIM), jnp.float32),
            pltpu.VMEM((CH, DIM), jnp.float32),
            pltpu.VMEM((CH, DIM), jnp.float32),
            pltpu.VMEM((CH, DIM), jnp.float32),
            pltpu.VMEM((4, LANES), jnp.float32),
            pltpu.SemaphoreType.DMA,
            pltpu.SemaphoreType.DMA,
            pltpu.SemaphoreType.DMA,
            pltpu.SemaphoreType.DMA,
            pltpu.SemaphoreType.DMA,
            pltpu.SemaphoreType.DMA,
            pltpu.SemaphoreType.DMA,
            pltpu.SemaphoreType.DMA,
        ],
    )
    return f(h, e, w, hn, en, jnp.asarray(_CONSTS))


def kernel(hidden_states, target_embeddings, embedding_weight):
    hn, en = _row_norms(hidden_states, target_embeddings)
    partials = _sc_pair_loss(hidden_states, target_embeddings,
                             embedding_weight, hn, en)
    sums = jnp.sum(partials, axis=(0, 2))
    inv = jnp.float32(1.0 / NUM_PAIRS)
    geo_gap = sums[0] * inv
    geo_loss = sums[1] * inv
    anc_gap = sums[2] * inv
    anc_loss = sums[3] * inv
    total = geo_loss + 0.5 * anc_loss
    raw_gap = geo_gap + 0.5 * anc_gap
    return (total, geo_loss, anc_loss, raw_gap)


# speedup vs baseline: 1.1764x; 1.1764x over previous
import jax
import jax.numpy as jnp
import numpy as np
from jax import lax
from jax.experimental import pallas as pl
from jax.experimental.pallas import tpu as pltpu
from jax.experimental.pallas import tpu_sc as plsc

BATCH = 8192
DIM = 1024
VOCAB = 100000
NUM_PAIRS = BATCH // 2
NC = 2
NS = 16
LANES = 16
NW = NC * NS
PT = NUM_PAIRS // NW
CH = 8
NCH = PT // CH
NSS = NCH // 2
NG = PT // LANES
NK = DIM // LANES
NMASK = BATCH // LANES
SIGMOID_SCALE = 10.0


_U32 = np.uint32


def _tf2x32(k1, k2, c1, c2):
    rot0 = (13, 15, 26, 6)
    rot1 = (17, 29, 16, 24)
    ks0 = _U32(k1)
    ks1 = _U32(k2)
    ks2 = _U32(ks0 ^ ks1 ^ _U32(0x1BD11BDA))
    x0 = c1.astype(_U32)
    x1 = c2.astype(_U32)
    with np.errstate(over="ignore"):
        x0 = x0 + ks0
        x1 = x1 + ks1

        def rounds(x0, x1, rots):
            for r in rots:
                x0 = (x0 + x1).astype(_U32)
                x1 = ((x1 << _U32(r)) | (x1 >> _U32(32 - r))).astype(_U32)
                x1 = x1 ^ x0
            return x0, x1

        for i, rots in enumerate((rot0, rot1, rot0, rot1, rot0)):
            x0, x1 = rounds(x0, x1, rots)
            ka, kb = ((ks1, ks2), (ks2, ks0), (ks0, ks1),
                      (ks1, ks2), (ks2, ks0))[i]
            x0 = (x0 + ka).astype(_U32)
            x1 = (x1 + kb + _U32(i + 1)).astype(_U32)
    return x0, x1


def _np_split(kpair, num):
    b1, b2 = _tf2x32(kpair[0], kpair[1], np.zeros(num, _U32),
                     np.arange(num, dtype=_U32))
    return np.stack([b1, b2], axis=1)


def _np_bits(kpair, size):
    b1, b2 = _tf2x32(kpair[0], kpair[1], np.zeros(size, _U32),
                     np.arange(size, dtype=_U32))
    return b1 ^ b2


def _pair_constants():
    kd = np.array([0, 42], _U32)
    ki, kj, ko, ke = _np_split(kd, 4)

    def bits2(kp):
        kk = _np_split(kp, 2)
        return _np_bits(kk[0], NUM_PAIRS), _np_bits(kk[1], NUM_PAIRS)

    hbi, lbi = bits2(ki)
    hbj, lbj = bits2(kj)
    hbo, lbo = bits2(ko)
    hbe, lbe = bits2(ke)
    span = _U32(VOCAB)
    with np.errstate(over="ignore"):
        mult = _U32(65536) % span
        mult = _U32(mult * mult) % span
        emb = (((hbe % span) * mult + (lbe % span)) % span).astype(np.int32)
    return np.concatenate([hbi, lbi, hbj, lbj, hbo, lbo,
                           emb.view(np.uint32)])


_CONSTS = _pair_constants()


def _norm_body(h_ref, e_ref, hn_ref, en_ref):
    h = h_ref[...]
    e = e_ref[...]
    hn_ref[...] = jnp.sqrt(jnp.sum(h * h, axis=-1))
    en_ref[...] = jnp.sqrt(jnp.sum(e * e, axis=-1))


def _row_norms(hidden, target):
    h3 = hidden.reshape(64, 128, DIM)
    e3 = target.reshape(64, 128, DIM)
    hn, en = pl.pallas_call(
        _norm_body,
        grid=(8,),
        in_specs=[
            pl.BlockSpec((8, 128, DIM), lambda i: (i, 0, 0)),
            pl.BlockSpec((8, 128, DIM), lambda i: (i, 0, 0)),
        ],
        out_specs=[
            pl.BlockSpec((8, 128), lambda i: (i, 0)),
            pl.BlockSpec((8, 128), lambda i: (i, 0)),
        ],
        out_shape=[
            jax.ShapeDtypeStruct((64, 128), jnp.float32),
            jax.ShapeDtypeStruct((64, 128), jnp.float32),
        ],
    )(h3, e3)
    return hn.reshape(BATCH), en.reshape(BATCH)


_GATHER_DNUMS = lax.GatherDimensionNumbers(
    offset_dims=(), collapsed_slice_dims=(0,), start_index_map=(0,))


def _lane_perm(v, idx):
    return lax.gather(v, idx.reshape(LANES, 1), _GATHER_DNUMS, (1,),
                      mode=lax.GatherScatterMode.PROMISE_IN_BOUNDS)


def _lane_allsum(v):
    lanes = lax.iota(jnp.int32, LANES)
    for k in (1, 2, 4, 8):
        v = v + _lane_perm(v, lanes ^ k)
    return v


def _rsqrt_newton(x):
    i = plsc.bitcast(x, jnp.int32)
    i = jnp.int32(0x5F3759DF) - lax.shift_right_arithmetic(i, 1)
    y = plsc.bitcast(i, jnp.float32)
    for _ in range(3):
        y = y * (1.5 - 0.5 * x * y * y)
    return y


def _sigmoid_sq(gap):
    s = 1.0 / (1.0 + jnp.exp(-SIGMOID_SCALE * gap))
    d = s - 0.5
    return d * d


def _randint_vec(hb, lb, span):
    mult = jnp.uint32(65536) % span
    mult = (mult * mult) % span
    off = ((hb % span) * mult + (lb % span)) % span
    return off.astype(jnp.int32)


def _gather_start(table, idx_slice, buf, sem):
    pltpu.make_async_copy(table.at[idx_slice], buf, sem).start()


def _gather_wait(table, idx_slice, buf, sem):
    pltpu.make_async_copy(table.at[idx_slice], buf, sem).wait()


def _sc_body(h_hbm, e_hbm, w_hbm, hn_hbm, en_hbm, cb_hbm,
             out_hbm,
             hn_v, en_v, vi_v, gi_v, gj_v, go_v, we_v, weu_v,
             hbi_v, lbi_v, hbj_v, lbj_v, hbo_v, lbo_v,
             ba0, bb0, bc0, bd0, ba1, bb1, bc1, bd1, psum_v,
             sa0, sb0, sc0, sd0, sa1, sb1, sc1, sd1):
    wid = lax.axis_index("s") * NC + lax.axis_index("c")
    base = wid * PT
    pltpu.sync_copy(hn_hbm, hn_v)
    pltpu.sync_copy(en_hbm, en_v)
    pltpu.sync_copy(cb_hbm.at[pl.ds(0 * NUM_PAIRS + base, PT)], hbi_v)
    pltpu.sync_copy(cb_hbm.at[pl.ds(1 * NUM_PAIRS + base, PT)], lbi_v)
    pltpu.sync_copy(cb_hbm.at[pl.ds(2 * NUM_PAIRS + base, PT)], hbj_v)
    pltpu.sync_copy(cb_hbm.at[pl.ds(3 * NUM_PAIRS + base, PT)], lbj_v)
    pltpu.sync_copy(cb_hbm.at[pl.ds(4 * NUM_PAIRS + base, PT)], hbo_v)
    pltpu.sync_copy(cb_hbm.at[pl.ds(5 * NUM_PAIRS + base, PT)], lbo_v)
    pltpu.sync_copy(cb_hbm.at[pl.ds(6 * NUM_PAIRS + base, PT)], weu_v)

    zeros = jnp.zeros((LANES,), jnp.float32)
    lanes = lax.iota(jnp.int32, LANES)
    last = jnp.full((LANES,), LANES - 1, jnp.int32)

    vi_v[pl.ds(0, LANES)] = jnp.zeros((LANES,), jnp.int32)

    def mask_body(i, off):
        sl = pl.ds(i * LANES, LANES)
        m = (hn_v[sl] > 1e-8) & (en_v[sl] > 1e-8)
        mi = m.astype(jnp.int32)
        cs = plsc.cumsum(mi)
        plsc.store_scatter(vi_v, [off + cs - 1], lanes + i * LANES, mask=m)
        return off + plsc.all_reduce_population_count(m)

    vb_vec = lax.fori_loop(0, NMASK, mask_body,
                           jnp.zeros((LANES,), jnp.int32))

    span_i = jnp.maximum(vb_vec, 1).astype(jnp.uint32)
    span_j = jnp.maximum(vb_vec - 1, 1).astype(jnp.uint32)

    def gen_body(g, carry):
        sl = pl.ds(g * LANES, LANES)
        ii = _randint_vec(hbi_v[sl], lbi_v[sl], span_i)
        jj = _randint_vec(hbj_v[sl], lbj_v[sl], span_j)
        jj = jj + (jj >= ii).astype(jnp.int32)
        oo = _randint_vec(hbo_v[sl], lbo_v[sl], span_i)
        gi_v[sl] = plsc.load_gather(vi_v, [ii])
        gj_v[sl] = plsc.load_gather(vi_v, [jj])
        go_v[sl] = plsc.load_gather(vi_v, [oo])
        we_v[sl] = plsc.bitcast(weu_v[sl], jnp.int32)
        return carry

    lax.fori_loop(0, NG, gen_body, 0)

    def gi_sl(c):
        return gi_v.at[pl.ds(c * CH, CH)]

    def gj_sl(c):
        return gj_v.at[pl.ds(c * CH, CH)]

    def go_sl(c):
        return go_v.at[pl.ds(c * CH, CH)]

    def we_sl(c):
        return we_v.at[pl.ds(c * CH, CH)]

    def geo_issue(c, ba, bb, bc, bd, sa, sb, sc, sd):
        _gather_start(h_hbm, gi_sl(c), ba, sa)
        _gather_start(h_hbm, gj_sl(c), bb, sb)
        _gather_start(e_hbm, gi_sl(c), bc, sc)
        _gather_start(e_hbm, gj_sl(c), bd, sd)

    def geo_wait(c, ba, bb, bc, bd, sa, sb, sc, sd):
        _gather_wait(h_hbm, gi_sl(c), ba, sa)
        _gather_wait(h_hbm, gj_sl(c), bb, sb)
        _gather_wait(e_hbm, gi_sl(c), bc, sc)
        _gather_wait(e_hbm, gj_sl(c), bd, sd)

    def geo_dots8(ba, bb, bc, bd, dot_h, dot_e, lane_base):
        for p in range(CH):
            def kbody(k, acc, _p=p):
                ah, ae = acc
                ko = k * LANES
                ah = ah + ba[_p, pl.ds(ko, LANES)] * bb[_p, pl.ds(ko, LANES)]
                ae = ae + bc[_p, pl.ds(ko, LANES)] * bd[_p, pl.ds(ko, LANES)]
                return ah, ae
            ah, ae = lax.fori_loop(0, NK, kbody, (zeros, zeros), unroll=8)
            sel = lanes == (lane_base + p)
            dot_h = jnp.where(sel, _lane_allsum(ah), dot_h)
            dot_e = jnp.where(sel, _lane_allsum(ae), dot_e)
        return dot_h, dot_e

    geo_issue(0, ba0, bb0, bc0, bd0, sa0, sb0, sc0, sd0)

    def geo_step(s, carry):
        s_abs, s_loss = carry
        c0 = 2 * s
        c1 = c0 + 1
        geo_issue(c1, ba1, bb1, bc1, bd1, sa1, sb1, sc1, sd1)
        geo_wait(c0, ba0, bb0, bc0, bd0, sa0, sb0, sc0, sd0)
        dot_h, dot_e = geo_dots8(ba0, bb0, bc0, bd0, zeros, zeros, 0)

        @pl.when(s < NSS - 1)
        def _():
            geo_issue(c0 + 2, ba0, bb0, bc0, bd0, sa0, sb0, sc0, sd0)

        geo_wait(c1, ba1, bb1, bc1, bd1, sa1, sb1, sc1, sd1)
        dot_h, dot_e = geo_dots8(ba1, bb1, bc1, bd1, dot_h, dot_e, CH)

        sl = pl.ds(s * LANES, LANES)
        gi_vec = gi_v[sl]
        gj_vec = gj_v[sl]
        hn_i = plsc.load_gather(hn_v, [gi_vec])
        hn_j = plsc.load_gather(hn_v, [gj_vec])
        en_i = plsc.load_gather(en_v, [gi_vec])
        en_j = plsc.load_gather(en_v, [gj_vec])
        sim_h = dot_h / jnp.maximum(hn_i * hn_j, 1e-8)
        sim_e = dot_e / jnp.maximum(en_i * en_j, 1e-8)
        gap = sim_h - sim_e
        return s_abs + jnp.abs(gap), s_loss + _sigmoid_sq(gap)

    g_abs, g_loss = lax.fori_loop(0, NSS, geo_step, (zeros, zeros))

    def anc_issue(c, ba, bb, bc, sa, sb, sc):
        _gather_start(h_hbm, go_sl(c), ba, sa)
        _gather_start(w_hbm, we_sl(c), bb, sb)
        _gather_start(e_hbm, go_sl(c), bc, sc)

    def anc_wait(c, ba, bb, bc, sa, sb, sc):
        _gather_wait(h_hbm, go_sl(c), ba, sa)
        _gather_wait(w_hbm, we_sl(c), bb, sb)
        _gather_wait(e_hbm, go_sl(c), bc, sc)

    def anc_dots8(ba, bb, bc, dot_hw, dot_ew, dot_ww, lane_base):
        for p in range(CH):
            def kbody(k, acc, _p=p):
                aw, ew, ww = acc
                ko = k * LANES
                wv = bb[_p, pl.ds(ko, LANES)]
                aw = aw + ba[_p, pl.ds(ko, LANES)] * wv
                ew = ew + bc[_p, pl.ds(ko, LANES)] * wv
                ww = ww + wv * wv
                return aw, ew, ww
            aw, ew, ww = lax.fori_loop(0, NK, kbody, (zeros, zeros, zeros),
                                       unroll=8)
            sel = lanes == (lane_base + p)
            dot_hw = jnp.where(sel, _lane_allsum(aw), dot_hw)
            dot_ew = jnp.where(sel, _lane_allsum(ew), dot_ew)
            dot_ww = jnp.where(sel, _lane_allsum(ww), dot_ww)
        return dot_hw, dot_ew, dot_ww

    anc_issue(0, ba0, bb0, bc0, sa0, sb0, sc0)

    def anc_step(s, carry):
        s_abs, s_loss = carry
        c0 = 2 * s
        c1 = c0 + 1
        anc_issue(c1, ba1, bb1, bc1, sa1, sb1, sc1)
        anc_wait(c0, ba0, bb0, bc0, sa0, sb0, sc0)
        dot_hw, dot_ew, dot_ww = anc_dots8(
            ba0, bb0, bc0, zeros, zeros, zeros, 0)

        @pl.when(s < NSS - 1)
        def _():
            anc_issue(c0 + 2, ba0, bb0, bc0, sa0, sb0, sc0)

        anc_wait(c1, ba1, bb1, bc1, sa1, sb1, sc1)
        dot_hw, dot_ew, dot_ww = anc_dots8(
            ba1, bb1, bc1, dot_hw, dot_ew, dot_ww, CH)

        sl = pl.ds(s * LANES, LANES)
        go_vec = go_v[sl]
        hn_o = plsc.load_gather(hn_v, [go_vec])
        en_o = plsc.load_gather(en_v, [go_vec])
        wn2 = jnp.maximum(dot_ww, 1e-30)
        wn = wn2 * _rsqrt_newton(wn2)
        sim_h = dot_hw / jnp.maximum(hn_o * wn, 1e-8)
        sim_e = dot_ew / jnp.maximum(en_o * wn, 1e-8)
        gap = sim_h - sim_e
        return s_abs + jnp.abs(gap), s_loss + _sigmoid_sq(gap)

    a_abs, a_loss = lax.fori_loop(0, NSS, anc_step, (zeros, zeros))

    psum_v[0, :] = g_abs
    psum_v[1, :] = g_loss
    psum_v[2, :] = a_abs
    psum_v[3, :] = a_loss
    pltpu.sync_copy(psum_v, out_hbm.at[wid])


def _sc_pair_loss(h, e, w, hn, en):
    mesh = plsc.VectorSubcoreMesh(core_axis_name="c", subcore_axis_name="s")
    f = pl.kernel(
        _sc_body,
        mesh=mesh,
        compiler_params=pltpu.CompilerParams(needs_layout_passes=False),
        out_type=jax.ShapeDtypeStruct((NW, 4, LANES), jnp.float32),
        scratch_types=[
            pltpu.VMEM((BATCH,), jnp.float32),
            pltpu.VMEM((BATCH,), jnp.float32),
            pltpu.VMEM((BATCH,), jnp.int32),
            pltpu.VMEM((PT,), jnp.int32),
            pltpu.VMEM((PT,), jnp.int32),
            pltpu.VMEM((PT,), jnp.int32),
            pltpu.VMEM((PT,), jnp.int32),
            pltpu.VMEM((PT,), jnp.uint32),
            pltpu.VMEM((PT,), jnp.uint32),
            pltpu.VMEM((PT,), jnp.uint32),
            pltpu.VMEM((PT,), jnp.uint32),
            pltpu.VMEM((PT,), jnp.uint32),
            pltpu.VMEM((PT,), jnp.uint32),
            pltpu.VMEM((PT,), jnp.uint32),
            pltpu.VMEM((CH, DIM), jnp.float32),
            pltpu.VMEM((CH, DIM), jnp.float32),
            pltpu.VMEM((CH, DIM), jnp.float32),
            pltpu.VMEM((CH, DIM), jnp.float32),
            pltpu.VMEM((CH, DIM), jnp.float32),
            pltpu.VMEM((CH, DIM), jnp.float32),
            pltpu.VMEM((CH, DIM), jnp.float32),
            pltpu.VMEM((CH, DIM), jnp.float32),
            pltpu.VMEM((4, LANES), jnp.float32),
            pltpu.SemaphoreType.DMA,
            pltpu.SemaphoreType.DMA,
            pltpu.SemaphoreType.DMA,
            pltpu.SemaphoreType.DMA,
            pltpu.SemaphoreType.DMA,
            pltpu.SemaphoreType.DMA,
            pltpu.SemaphoreType.DMA,
            pltpu.SemaphoreType.DMA,
        ],
    )
    return f(h, e, w, hn, en, jnp.asarray(_CONSTS))


def kernel(hidden_states, target_embeddings, embedding_weight):
    hn, en = _row_norms(hidden_states, target_embeddings)
    partials = _sc_pair_loss(hidden_states, target_embeddings,
                             embedding_weight, hn, en)
    sums = jnp.sum(partials, axis=(0, 2))
    inv = jnp.float32(1.0 / NUM_PAIRS)
    geo_gap = sums[0] * inv
    geo_loss = sums[1] * inv
    anc_gap = sums[2] * inv
    anc_loss = sums[3] * inv
    total = geo_loss + 0.5 * anc_loss
    raw_gap = geo_gap + 0.5 * anc_gap
    return (total, geo_loss, anc_loss, raw_gap)


# speedup vs baseline: 1.1795x; 1.0027x over previous
import jax
import jax.numpy as jnp
import numpy as np
from jax import lax
from jax.experimental import pallas as pl
from jax.experimental.pallas import tpu as pltpu
from jax.experimental.pallas import tpu_sc as plsc

BATCH = 8192
DIM = 1024
VOCAB = 100000
NUM_PAIRS = BATCH // 2
NC = 2
NS = 16
LANES = 16
NW = NC * NS
PT = NUM_PAIRS // NW
CH = 8
NCH = PT // CH
NSS = NCH // 2
NG = PT // LANES
NK = DIM // LANES
NMASK = BATCH // LANES
SIGMOID_SCALE = 10.0


_U32 = np.uint32


def _tf2x32(k1, k2, c1, c2):
    rot0 = (13, 15, 26, 6)
    rot1 = (17, 29, 16, 24)
    ks0 = _U32(k1)
    ks1 = _U32(k2)
    ks2 = _U32(ks0 ^ ks1 ^ _U32(0x1BD11BDA))
    x0 = c1.astype(_U32)
    x1 = c2.astype(_U32)
    with np.errstate(over="ignore"):
        x0 = x0 + ks0
        x1 = x1 + ks1

        def rounds(x0, x1, rots):
            for r in rots:
                x0 = (x0 + x1).astype(_U32)
                x1 = ((x1 << _U32(r)) | (x1 >> _U32(32 - r))).astype(_U32)
                x1 = x1 ^ x0
            return x0, x1

        for i, rots in enumerate((rot0, rot1, rot0, rot1, rot0)):
            x0, x1 = rounds(x0, x1, rots)
            ka, kb = ((ks1, ks2), (ks2, ks0), (ks0, ks1),
                      (ks1, ks2), (ks2, ks0))[i]
            x0 = (x0 + ka).astype(_U32)
            x1 = (x1 + kb + _U32(i + 1)).astype(_U32)
    return x0, x1


def _np_split(kpair, num):
    b1, b2 = _tf2x32(kpair[0], kpair[1], np.zeros(num, _U32),
                     np.arange(num, dtype=_U32))
    return np.stack([b1, b2], axis=1)


def _np_bits(kpair, size):
    b1, b2 = _tf2x32(kpair[0], kpair[1], np.zeros(size, _U32),
                     np.arange(size, dtype=_U32))
    return b1 ^ b2


def _pair_constants():
    kd = np.array([0, 42], _U32)
    ki, kj, ko, ke = _np_split(kd, 4)

    def bits2(kp):
        kk = _np_split(kp, 2)
        return _np_bits(kk[0], NUM_PAIRS), _np_bits(kk[1], NUM_PAIRS)

    hbi, lbi = bits2(ki)
    hbj, lbj = bits2(kj)
    hbo, lbo = bits2(ko)
    hbe, lbe = bits2(ke)
    span = _U32(VOCAB)
    with np.errstate(over="ignore"):
        mult = _U32(65536) % span
        mult = _U32(mult * mult) % span
        emb = (((hbe % span) * mult + (lbe % span)) % span).astype(np.int32)
    return np.concatenate([hbi, lbi, hbj, lbj, hbo, lbo,
                           emb.view(np.uint32)])


_CONSTS = _pair_constants()


def _norm_body(h_ref, e_ref, hn_ref, en_ref):
    h = h_ref[...]
    e = e_ref[...]
    hn_ref[...] = jnp.sqrt(jnp.sum(h * h, axis=-1))
    en_ref[...] = jnp.sqrt(jnp.sum(e * e, axis=-1))


def _row_norms(hidden, target):
    h3 = hidden.reshape(64, 128, DIM)
    e3 = target.reshape(64, 128, DIM)
    hn, en = pl.pallas_call(
        _norm_body,
        grid=(8,),
        in_specs=[
            pl.BlockSpec((8, 128, DIM), lambda i: (i, 0, 0)),
            pl.BlockSpec((8, 128, DIM), lambda i: (i, 0, 0)),
        ],
        out_specs=[
            pl.BlockSpec((8, 128), lambda i: (i, 0)),
            pl.BlockSpec((8, 128), lambda i: (i, 0)),
        ],
        out_shape=[
            jax.ShapeDtypeStruct((64, 128), jnp.float32),
            jax.ShapeDtypeStruct((64, 128), jnp.float32),
        ],
    )(h3, e3)
    return hn.reshape(BATCH), en.reshape(BATCH)


_GATHER_DNUMS = lax.GatherDimensionNumbers(
    offset_dims=(), collapsed_slice_dims=(0,), start_index_map=(0,))


def _lane_perm(v, idx):
    return lax.gather(v, idx.reshape(LANES, 1), _GATHER_DNUMS, (1,),
                      mode=lax.GatherScatterMode.PROMISE_IN_BOUNDS)


def _lane_allsum(v):
    lanes = lax.iota(jnp.int32, LANES)
    for k in (1, 2, 4, 8):
        v = v + _lane_perm(v, lanes ^ k)
    return v


def _rsqrt_newton(x):
    i = plsc.bitcast(x, jnp.int32)
    i = jnp.int32(0x5F3759DF) - lax.shift_right_arithmetic(i, 1)
    y = plsc.bitcast(i, jnp.float32)
    for _ in range(3):
        y = y * (1.5 - 0.5 * x * y * y)
    return y


def _sigmoid_sq(gap):
    s = 1.0 / (1.0 + jnp.exp(-SIGMOID_SCALE * gap))
    d = s - 0.5
    return d * d


def _randint_vec(hb, lb, span):
    mult = jnp.uint32(65536) % span
    mult = (mult * mult) % span
    off = ((hb % span) * mult + (lb % span)) % span
    return off.astype(jnp.int32)


def _gather_start(table, idx_slice, buf, sem):
    pltpu.make_async_copy(table.at[idx_slice], buf, sem).start()


def _gather_wait(table, idx_slice, buf, sem):
    pltpu.make_async_copy(table.at[idx_slice], buf, sem).wait()


def _sc_body(h_hbm, e_hbm, w_hbm, hn_hbm, en_hbm, cb_hbm,
             out_hbm,
             hn_v, en_v, vi_v, gi_v, gj_v, go_v, we_v, weu_v,
             hbi_v, lbi_v, hbj_v, lbj_v, hbo_v, lbo_v,
             ba0, bb0, bc0, bd0, ba1, bb1, bc1, bd1, psum_v,
             sa0, sb0, sc0, sd0, sa1, sb1, sc1, sd1):
    wid = lax.axis_index("s") * NC + lax.axis_index("c")
    base = wid * PT
    pltpu.sync_copy(hn_hbm, hn_v)
    pltpu.sync_copy(en_hbm, en_v)
    pltpu.sync_copy(cb_hbm.at[pl.ds(0 * NUM_PAIRS + base, PT)], hbi_v)
    pltpu.sync_copy(cb_hbm.at[pl.ds(1 * NUM_PAIRS + base, PT)], lbi_v)
    pltpu.sync_copy(cb_hbm.at[pl.ds(2 * NUM_PAIRS + base, PT)], hbj_v)
    pltpu.sync_copy(cb_hbm.at[pl.ds(3 * NUM_PAIRS + base, PT)], lbj_v)
    pltpu.sync_copy(cb_hbm.at[pl.ds(4 * NUM_PAIRS + base, PT)], hbo_v)
    pltpu.sync_copy(cb_hbm.at[pl.ds(5 * NUM_PAIRS + base, PT)], lbo_v)
    pltpu.sync_copy(cb_hbm.at[pl.ds(6 * NUM_PAIRS + base, PT)], weu_v)

    zeros = jnp.zeros((LANES,), jnp.float32)
    lanes = lax.iota(jnp.int32, LANES)
    last = jnp.full((LANES,), LANES - 1, jnp.int32)

    vi_v[pl.ds(0, LANES)] = jnp.zeros((LANES,), jnp.int32)

    def mask_body(i, off):
        sl = pl.ds(i * LANES, LANES)
        m = (hn_v[sl] > 1e-8) & (en_v[sl] > 1e-8)
        mi = m.astype(jnp.int32)
        cs = plsc.cumsum(mi)
        plsc.store_scatter(vi_v, [off + cs - 1], lanes + i * LANES, mask=m)
        return off + plsc.all_reduce_population_count(m)

    vb_vec = lax.fori_loop(0, NMASK, mask_body,
                           jnp.zeros((LANES,), jnp.int32))

    span_i = jnp.maximum(vb_vec, 1).astype(jnp.uint32)
    span_j = jnp.maximum(vb_vec - 1, 1).astype(jnp.uint32)

    def gen_body(g, carry):
        sl = pl.ds(g * LANES, LANES)
        ii = _randint_vec(hbi_v[sl], lbi_v[sl], span_i)
        jj = _randint_vec(hbj_v[sl], lbj_v[sl], span_j)
        jj = jj + (jj >= ii).astype(jnp.int32)
        oo = _randint_vec(hbo_v[sl], lbo_v[sl], span_i)
        gi_v[sl] = plsc.load_gather(vi_v, [ii])
        gj_v[sl] = plsc.load_gather(vi_v, [jj])
        go_v[sl] = plsc.load_gather(vi_v, [oo])
        we_v[sl] = plsc.bitcast(weu_v[sl], jnp.int32)
        return carry

    lax.fori_loop(0, NG, gen_body, 0)

    def gi_sl(c):
        return gi_v.at[pl.ds(c * CH, CH)]

    def gj_sl(c):
        return gj_v.at[pl.ds(c * CH, CH)]

    def go_sl(c):
        return go_v.at[pl.ds(c * CH, CH)]

    def we_sl(c):
        return we_v.at[pl.ds(c * CH, CH)]

    def geo_issue(c, ba, bb, bc, bd, sa, sb, sc, sd):
        _gather_start(h_hbm, gi_sl(c), ba, sa)
        _gather_start(h_hbm, gj_sl(c), bb, sb)
        _gather_start(e_hbm, gi_sl(c), bc, sc)
        _gather_start(e_hbm, gj_sl(c), bd, sd)

    def geo_wait(c, ba, bb, bc, bd, sa, sb, sc, sd):
        _gather_wait(h_hbm, gi_sl(c), ba, sa)
        _gather_wait(h_hbm, gj_sl(c), bb, sb)
        _gather_wait(e_hbm, gi_sl(c), bc, sc)
        _gather_wait(e_hbm, gj_sl(c), bd, sd)

    def geo_dots8(ba, bb, bc, bd, dot_h, dot_e, lane_base):
        for p in range(CH):
            def kbody(k, acc, _p=p):
                ah, ae = acc
                ko = k * LANES
                ah = ah + ba[_p, pl.ds(ko, LANES)] * bb[_p, pl.ds(ko, LANES)]
                ae = ae + bc[_p, pl.ds(ko, LANES)] * bd[_p, pl.ds(ko, LANES)]
                return ah, ae
            ah, ae = lax.fori_loop(0, NK, kbody, (zeros, zeros), unroll=8)
            sel = lanes == (lane_base + p)
            dot_h = jnp.where(sel, _lane_allsum(ah), dot_h)
            dot_e = jnp.where(sel, _lane_allsum(ae), dot_e)
        return dot_h, dot_e

    geo_issue(0, ba0, bb0, bc0, bd0, sa0, sb0, sc0, sd0)

    def geo_step(s, carry):
        s_abs, s_loss = carry
        c0 = 2 * s
        c1 = c0 + 1
        geo_issue(c1, ba1, bb1, bc1, bd1, sa1, sb1, sc1, sd1)
        geo_wait(c0, ba0, bb0, bc0, bd0, sa0, sb0, sc0, sd0)
        dot_h, dot_e = geo_dots8(ba0, bb0, bc0, bd0, zeros, zeros, 0)

        @pl.when(s < NSS - 1)
        def _():
            geo_issue(c0 + 2, ba0, bb0, bc0, bd0, sa0, sb0, sc0, sd0)

        geo_wait(c1, ba1, bb1, bc1, bd1, sa1, sb1, sc1, sd1)
        dot_h, dot_e = geo_dots8(ba1, bb1, bc1, bd1, dot_h, dot_e, CH)

        sl = pl.ds(s * LANES, LANES)
        gi_vec = gi_v[sl]
        gj_vec = gj_v[sl]
        hn_i = plsc.load_gather(hn_v, [gi_vec])
        hn_j = plsc.load_gather(hn_v, [gj_vec])
        en_i = plsc.load_gather(en_v, [gi_vec])
        en_j = plsc.load_gather(en_v, [gj_vec])
        sim_h = dot_h / jnp.maximum(hn_i * hn_j, 1e-8)
        sim_e = dot_e / jnp.maximum(en_i * en_j, 1e-8)
        gap = sim_h - sim_e
        return s_abs + jnp.abs(gap), s_loss + _sigmoid_sq(gap)

    g_abs, g_loss = lax.fori_loop(0, NSS, geo_step, (zeros, zeros))

    def anc_issue(c, ba, bb, bc, sa, sb, sc):
        _gather_start(h_hbm, go_sl(c), ba, sa)
        _gather_start(w_hbm, we_sl(c), bb, sb)
        _gather_start(e_hbm, go_sl(c), bc, sc)

    def anc_wait(c, ba, bb, bc, sa, sb, sc):
        _gather_wait(h_hbm, go_sl(c), ba, sa)
        _gather_wait(w_hbm, we_sl(c), bb, sb)
        _gather_wait(e_hbm, go_sl(c), bc, sc)

    def anc_dots8(ba, bb, bc, dot_hw, dot_ew, dot_ww, lane_base):
        for p in range(CH):
            def kbody(k, acc, _p=p):
                aw, ew, ww = acc
                ko = k * LANES
                wv = bb[_p, pl.ds(ko, LANES)]
                aw = aw + ba[_p, pl.ds(ko, LANES)] * wv
                ew = ew + bc[_p, pl.ds(ko, LANES)] * wv
                ww = ww + wv * wv
                return aw, ew, ww
            aw, ew, ww = lax.fori_loop(0, NK, kbody, (zeros, zeros, zeros),
                                       unroll=8)
            sel = lanes == (lane_base + p)
            dot_hw = jnp.where(sel, _lane_allsum(aw), dot_hw)
            dot_ew = jnp.where(sel, _lane_allsum(ew), dot_ew)
            dot_ww = jnp.where(sel, _lane_allsum(ww), dot_ww)
        return dot_hw, dot_ew, dot_ww

    anc_issue(0, ba0, bb0, bc0, sa0, sb0, sc0)

    def anc_step(s, carry):
        s_abs, s_loss = carry
        c0 = 2 * s
        c1 = c0 + 1
        anc_issue(c1, ba1, bb1, bc1, sa1, sb1, sc1)
        anc_wait(c0, ba0, bb0, bc0, sa0, sb0, sc0)
        dot_hw, dot_ew, dot_ww = anc_dots8(
            ba0, bb0, bc0, zeros, zeros, zeros, 0)

        @pl.when(s < NSS - 1)
        def _():
            anc_issue(c0 + 2, ba0, bb0, bc0, sa0, sb0, sc0)

        anc_wait(c1, ba1, bb1, bc1, sa1, sb1, sc1)
        dot_hw, dot_ew, dot_ww = anc_dots8(
            ba1, bb1, bc1, dot_hw, dot_ew, dot_ww, CH)

        sl = pl.ds(s * LANES, LANES)
        go_vec = go_v[sl]
        hn_o = plsc.load_gather(hn_v, [go_vec])
        en_o = plsc.load_gather(en_v, [go_vec])
        wn2 = jnp.maximum(dot_ww, 1e-30)
        wn = wn2 * _rsqrt_newton(wn2)
        sim_h = dot_hw / jnp.maximum(hn_o * wn, 1e-8)
        sim_e = dot_ew / jnp.maximum(en_o * wn, 1e-8)
        gap = sim_h - sim_e
        return s_abs + jnp.abs(gap), s_loss + _sigmoid_sq(gap)

    a_abs, a_loss = lax.fori_loop(0, NSS, anc_step, (zeros, zeros))

    psum_v[0, :] = g_abs
    psum_v[1, :] = g_loss
    psum_v[2, :] = a_abs
    psum_v[3, :] = a_loss
    pltpu.sync_copy(psum_v, out_hbm.at[wid])


def _sc_pair_loss(h, e, w, hn, en):
    mesh = plsc.VectorSubcoreMesh(core_axis_name="c", subcore_axis_name="s")
    f = pl.kernel(
        _sc_body,
        mesh=mesh,
        compiler_params=pltpu.CompilerParams(needs_layout_passes=False,
                                             skip_device_barrier=True),
        out_type=jax.ShapeDtypeStruct((NW, 4, LANES), jnp.float32),
        scratch_types=[
            pltpu.VMEM((BATCH,), jnp.float32),
            pltpu.VMEM((BATCH,), jnp.float32),
            pltpu.VMEM((BATCH,), jnp.int32),
            pltpu.VMEM((PT,), jnp.int32),
            pltpu.VMEM((PT,), jnp.int32),
            pltpu.VMEM((PT,), jnp.int32),
            pltpu.VMEM((PT,), jnp.int32),
            pltpu.VMEM((PT,), jnp.uint32),
            pltpu.VMEM((PT,), jnp.uint32),
            pltpu.VMEM((PT,), jnp.uint32),
            pltpu.VMEM((PT,), jnp.uint32),
            pltpu.VMEM((PT,), jnp.uint32),
            pltpu.VMEM((PT,), jnp.uint32),
            pltpu.VMEM((PT,), jnp.uint32),
            pltpu.VMEM((CH, DIM), jnp.float32),
            pltpu.VMEM((CH, DIM), jnp.float32),
            pltpu.VMEM((CH, DIM), jnp.float32),
            pltpu.VMEM((CH, DIM), jnp.float32),
            pltpu.VMEM((CH, DIM), jnp.float32),
            pltpu.VMEM((CH, DIM), jnp.float32),
            pltpu.VMEM((CH, DIM), jnp.float32),
            pltpu.VMEM((CH, DIM), jnp.float32),
            pltpu.VMEM((4, LANES), jnp.float32),
            pltpu.SemaphoreType.DMA,
            pltpu.SemaphoreType.DMA,
            pltpu.SemaphoreType.DMA,
            pltpu.SemaphoreType.DMA,
            pltpu.SemaphoreType.DMA,
            pltpu.SemaphoreType.DMA,
            pltpu.SemaphoreType.DMA,
            pltpu.SemaphoreType.DMA,
        ],
    )
    return f(h, e, w, hn, en, jnp.asarray(_CONSTS))


def kernel(hidden_states, target_embeddings, embedding_weight):
    hn, en = _row_norms(hidden_states, target_embeddings)
    partials = _sc_pair_loss(hidden_states, target_embeddings,
                             embedding_weight, hn, en)
    sums = jnp.sum(partials, axis=(0, 2))
    inv = jnp.float32(1.0 / NUM_PAIRS)
    geo_gap = sums[0] * inv
    geo_loss = sums[1] * inv
    anc_gap = sums[2] * inv
    anc_loss = sums[3] * inv
    total = geo_loss + 0.5 * anc_loss
    raw_gap = geo_gap + 0.5 * anc_gap
    return (total, geo_loss, anc_loss, raw_gap)


# speedup vs baseline: 1.2725x; 1.0788x over previous
import jax
import jax.numpy as jnp
import numpy as np
from jax import lax
from jax.experimental import pallas as pl
from jax.experimental.pallas import tpu as pltpu
from jax.experimental.pallas import tpu_sc as plsc

BATCH = 8192
DIM = 1024
VOCAB = 100000
NUM_PAIRS = BATCH // 2
NC = 2
NS = 16
LANES = 16
NW = NC * NS
PT = NUM_PAIRS // NW
CH = 4
NCH = PT // CH
NSS = NCH // 4
NG = PT // LANES
NK = DIM // LANES
NMASK = BATCH // LANES
SIGMOID_SCALE = 10.0


_U32 = np.uint32


def _tf2x32(k1, k2, c1, c2):
    rot0 = (13, 15, 26, 6)
    rot1 = (17, 29, 16, 24)
    ks0 = _U32(k1)
    ks1 = _U32(k2)
    ks2 = _U32(ks0 ^ ks1 ^ _U32(0x1BD11BDA))
    x0 = c1.astype(_U32)
    x1 = c2.astype(_U32)
    with np.errstate(over="ignore"):
        x0 = x0 + ks0
        x1 = x1 + ks1

        def rounds(x0, x1, rots):
            for r in rots:
                x0 = (x0 + x1).astype(_U32)
                x1 = ((x1 << _U32(r)) | (x1 >> _U32(32 - r))).astype(_U32)
                x1 = x1 ^ x0
            return x0, x1

        for i, rots in enumerate((rot0, rot1, rot0, rot1, rot0)):
            x0, x1 = rounds(x0, x1, rots)
            ka, kb = ((ks1, ks2), (ks2, ks0), (ks0, ks1),
                      (ks1, ks2), (ks2, ks0))[i]
            x0 = (x0 + ka).astype(_U32)
            x1 = (x1 + kb + _U32(i + 1)).astype(_U32)
    return x0, x1


def _np_split(kpair, num):
    b1, b2 = _tf2x32(kpair[0], kpair[1], np.zeros(num, _U32),
                     np.arange(num, dtype=_U32))
    return np.stack([b1, b2], axis=1)


def _np_bits(kpair, size):
    b1, b2 = _tf2x32(kpair[0], kpair[1], np.zeros(size, _U32),
                     np.arange(size, dtype=_U32))
    return b1 ^ b2


def _pair_constants():
    kd = np.array([0, 42], _U32)
    ki, kj, ko, ke = _np_split(kd, 4)

    def bits2(kp):
        kk = _np_split(kp, 2)
        return _np_bits(kk[0], NUM_PAIRS), _np_bits(kk[1], NUM_PAIRS)

    hbi, lbi = bits2(ki)
    hbj, lbj = bits2(kj)
    hbo, lbo = bits2(ko)
    hbe, lbe = bits2(ke)
    span = _U32(VOCAB)
    with np.errstate(over="ignore"):
        mult = _U32(65536) % span
        mult = _U32(mult * mult) % span
        emb = (((hbe % span) * mult + (lbe % span)) % span).astype(np.int32)
    return np.concatenate([hbi, lbi, hbj, lbj, hbo, lbo,
                           emb.view(np.uint32)])


_CONSTS = _pair_constants()


def _norm_body(h_ref, e_ref, hn_ref, en_ref):
    h = h_ref[...]
    e = e_ref[...]
    hn_ref[...] = jnp.sqrt(jnp.sum(h * h, axis=-1))
    en_ref[...] = jnp.sqrt(jnp.sum(e * e, axis=-1))


def _row_norms(hidden, target):
    h3 = hidden.reshape(64, 128, DIM)
    e3 = target.reshape(64, 128, DIM)
    hn, en = pl.pallas_call(
        _norm_body,
        grid=(8,),
        in_specs=[
            pl.BlockSpec((8, 128, DIM), lambda i: (i, 0, 0)),
            pl.BlockSpec((8, 128, DIM), lambda i: (i, 0, 0)),
        ],
        out_specs=[
            pl.BlockSpec((8, 128), lambda i: (i, 0)),
            pl.BlockSpec((8, 128), lambda i: (i, 0)),
        ],
        out_shape=[
            jax.ShapeDtypeStruct((64, 128), jnp.float32),
            jax.ShapeDtypeStruct((64, 128), jnp.float32),
        ],
    )(h3, e3)
    return hn.reshape(BATCH), en.reshape(BATCH)


_GATHER_DNUMS = lax.GatherDimensionNumbers(
    offset_dims=(), collapsed_slice_dims=(0,), start_index_map=(0,))


def _lane_perm(v, idx):
    return lax.gather(v, idx.reshape(LANES, 1), _GATHER_DNUMS, (1,),
                      mode=lax.GatherScatterMode.PROMISE_IN_BOUNDS)


def _lane_allsum(v):
    lanes = lax.iota(jnp.int32, LANES)
    for k in (1, 2, 4, 8):
        v = v + _lane_perm(v, lanes ^ k)
    return v


def _rsqrt_newton(x):
    i = plsc.bitcast(x, jnp.int32)
    i = jnp.int32(0x5F3759DF) - lax.shift_right_arithmetic(i, 1)
    y = plsc.bitcast(i, jnp.float32)
    for _ in range(3):
        y = y * (1.5 - 0.5 * x * y * y)
    return y


def _sigmoid_sq(gap):
    s = 1.0 / (1.0 + jnp.exp(-SIGMOID_SCALE * gap))
    d = s - 0.5
    return d * d


def _randint_vec(hb, lb, span):
    mult = jnp.uint32(65536) % span
    mult = (mult * mult) % span
    off = ((hb % span) * mult + (lb % span)) % span
    return off.astype(jnp.int32)


def _gather_start(table, idx_slice, buf, sem):
    pltpu.make_async_copy(table.at[idx_slice], buf, sem).start()


def _gather_wait(table, idx_slice, buf, sem):
    pltpu.make_async_copy(table.at[idx_slice], buf, sem).wait()


def _sc_body(h_hbm, e_hbm, w_hbm, hn_hbm, en_hbm, cb_hbm,
             out_hbm,
             hn_v, en_v, vi_v, gi_v, gj_v, go_v, we_v, weu_v,
             gi_p, gj_p, go_p, we_p,
             hbi_v, lbi_v, hbj_v, lbj_v, hbo_v, lbo_v,
             ba0, ba1, ba2, ba3, bb0, bb1, bb2, bb3,
             bc0, bc1, bc2, bc3, bd0, bd1, bd2, bd3, psum_v,
             sm0, sm1, sm2, sm3):
    wid = lax.axis_index("s") * NC + lax.axis_index("c")
    base = wid * PT
    pltpu.sync_copy(hn_hbm, hn_v)
    pltpu.sync_copy(en_hbm, en_v)
    pltpu.sync_copy(cb_hbm.at[pl.ds(0 * NUM_PAIRS + base, PT)], hbi_v)
    pltpu.sync_copy(cb_hbm.at[pl.ds(1 * NUM_PAIRS + base, PT)], lbi_v)
    pltpu.sync_copy(cb_hbm.at[pl.ds(2 * NUM_PAIRS + base, PT)], hbj_v)
    pltpu.sync_copy(cb_hbm.at[pl.ds(3 * NUM_PAIRS + base, PT)], lbj_v)
    pltpu.sync_copy(cb_hbm.at[pl.ds(4 * NUM_PAIRS + base, PT)], hbo_v)
    pltpu.sync_copy(cb_hbm.at[pl.ds(5 * NUM_PAIRS + base, PT)], lbo_v)
    pltpu.sync_copy(cb_hbm.at[pl.ds(6 * NUM_PAIRS + base, PT)], weu_v)

    zeros = jnp.zeros((LANES,), jnp.float32)
    lanes = lax.iota(jnp.int32, LANES)
    last = jnp.full((LANES,), LANES - 1, jnp.int32)

    vi_v[pl.ds(0, LANES)] = jnp.zeros((LANES,), jnp.int32)

    def mask_body(i, off):
        sl = pl.ds(i * LANES, LANES)
        m = (hn_v[sl] > 1e-8) & (en_v[sl] > 1e-8)
        mi = m.astype(jnp.int32)
        cs = plsc.cumsum(mi)
        plsc.store_scatter(vi_v, [off + cs - 1], lanes + i * LANES, mask=m)
        return off + plsc.all_reduce_population_count(m)

    vb_vec = lax.fori_loop(0, NMASK, mask_body,
                           jnp.zeros((LANES,), jnp.int32))

    span_i = jnp.maximum(vb_vec, 1).astype(jnp.uint32)
    span_j = jnp.maximum(vb_vec - 1, 1).astype(jnp.uint32)

    def gen_body(g, carry):
        sl = pl.ds(g * LANES, LANES)
        ii = _randint_vec(hbi_v[sl], lbi_v[sl], span_i)
        jj = _randint_vec(hbj_v[sl], lbj_v[sl], span_j)
        jj = jj + (jj >= ii).astype(jnp.int32)
        oo = _randint_vec(hbo_v[sl], lbo_v[sl], span_i)
        giv = plsc.load_gather(vi_v, [ii])
        gjv = plsc.load_gather(vi_v, [jj])
        gov = plsc.load_gather(vi_v, [oo])
        wev = plsc.bitcast(weu_v[sl], jnp.int32)
        gi_v[sl] = giv
        gj_v[sl] = gjv
        go_v[sl] = gov
        we_v[sl] = wev
        pos = (lanes & 3) + ((lanes >> 2) * 8) + g * 32
        plsc.store_scatter(gi_p, [pos], giv)
        plsc.store_scatter(gj_p, [pos], gjv)
        plsc.store_scatter(go_p, [pos], gov)
        plsc.store_scatter(we_p, [pos], wev)
        return carry

    lax.fori_loop(0, NG, gen_body, 0)

    def idx4(ref, c):
        return ref.at[pl.ds(c * 8, CH)]

    bufs_a = (ba0, ba1, ba2, ba3)
    bufs_b = (bb0, bb1, bb2, bb3)
    bufs_c = (bc0, bc1, bc2, bc3)
    bufs_d = (bd0, bd1, bd2, bd3)
    sems = (sm0, sm1, sm2, sm3)

    def geo_issue(c, j):
        _gather_start(h_hbm, idx4(gi_p, c), bufs_a[j], sems[j])
        _gather_start(h_hbm, idx4(gj_p, c), bufs_b[j], sems[j])
        _gather_start(e_hbm, idx4(gi_p, c), bufs_c[j], sems[j])
        _gather_start(e_hbm, idx4(gj_p, c), bufs_d[j], sems[j])

    def geo_wait(c, j):
        _gather_wait(h_hbm, idx4(gi_p, c), bufs_a[j], sems[j])
        _gather_wait(h_hbm, idx4(gj_p, c), bufs_b[j], sems[j])
        _gather_wait(e_hbm, idx4(gi_p, c), bufs_c[j], sems[j])
        _gather_wait(e_hbm, idx4(gj_p, c), bufs_d[j], sems[j])

    for j in range(4):
        geo_issue(j, j)

    def geo_step(s, carry):
        s_abs, s_loss = carry
        dot_h = zeros
        dot_e = zeros
        for j in range(4):
            c = 4 * s + j
            geo_wait(c, j)
            ba, bb, bc, bd = bufs_a[j], bufs_b[j], bufs_c[j], bufs_d[j]
            for p in range(CH):
                def kbody(k, acc, _p=p, _ba=ba, _bb=bb, _bc=bc, _bd=bd):
                    ah, ae = acc
                    ko = k * LANES
                    ah = ah + _ba[_p, pl.ds(ko, LANES)] * _bb[_p, pl.ds(ko, LANES)]
                    ae = ae + _bc[_p, pl.ds(ko, LANES)] * _bd[_p, pl.ds(ko, LANES)]
                    return ah, ae
                ah, ae = lax.fori_loop(0, NK, kbody, (zeros, zeros), unroll=8)
                sel = lanes == (4 * j + p)
                dot_h = jnp.where(sel, _lane_allsum(ah), dot_h)
                dot_e = jnp.where(sel, _lane_allsum(ae), dot_e)

            @pl.when(s < NSS - 1)
            def _(c=c, j=j):
                geo_issue(c + 4, j)

        sl = pl.ds(s * LANES, LANES)
        gi_vec = gi_v[sl]
        gj_vec = gj_v[sl]
        hn_i = plsc.load_gather(hn_v, [gi_vec])
        hn_j = plsc.load_gather(hn_v, [gj_vec])
        en_i = plsc.load_gather(en_v, [gi_vec])
        en_j = plsc.load_gather(en_v, [gj_vec])
        sim_h = dot_h / jnp.maximum(hn_i * hn_j, 1e-8)
        sim_e = dot_e / jnp.maximum(en_i * en_j, 1e-8)
        gap = sim_h - sim_e
        return s_abs + jnp.abs(gap), s_loss + _sigmoid_sq(gap)

    g_abs, g_loss = lax.fori_loop(0, NSS, geo_step, (zeros, zeros))

    def anc_issue(c, j):
        _gather_start(h_hbm, idx4(go_p, c), bufs_a[j], sems[j])
        _gather_start(w_hbm, idx4(we_p, c), bufs_b[j], sems[j])
        _gather_start(e_hbm, idx4(go_p, c), bufs_c[j], sems[j])

    def anc_wait(c, j):
        _gather_wait(h_hbm, idx4(go_p, c), bufs_a[j], sems[j])
        _gather_wait(w_hbm, idx4(we_p, c), bufs_b[j], sems[j])
        _gather_wait(e_hbm, idx4(go_p, c), bufs_c[j], sems[j])

    for j in range(4):
        anc_issue(j, j)

    def anc_step(s, carry):
        s_abs, s_loss = carry
        dot_hw = zeros
        dot_ew = zeros
        dot_ww = zeros
        for j in range(4):
            c = 4 * s + j
            anc_wait(c, j)
            ba, bb, bc = bufs_a[j], bufs_b[j], bufs_c[j]
            for p in range(CH):
                def kbody(k, acc, _p=p, _ba=ba, _bb=bb, _bc=bc):
                    aw, ew, ww = acc
                    ko = k * LANES
                    wv = _bb[_p, pl.ds(ko, LANES)]
                    aw = aw + _ba[_p, pl.ds(ko, LANES)] * wv
                    ew = ew + _bc[_p, pl.ds(ko, LANES)] * wv
                    ww = ww + wv * wv
                    return aw, ew, ww
                aw, ew, ww = lax.fori_loop(0, NK, kbody,
                                           (zeros, zeros, zeros), unroll=8)
                sel = lanes == (4 * j + p)
                dot_hw = jnp.where(sel, _lane_allsum(aw), dot_hw)
                dot_ew = jnp.where(sel, _lane_allsum(ew), dot_ew)
                dot_ww = jnp.where(sel, _lane_allsum(ww), dot_ww)

            @pl.when(s < NSS - 1)
            def _(c=c, j=j):
                anc_issue(c + 4, j)

        sl = pl.ds(s * LANES, LANES)
        go_vec = go_v[sl]
        hn_o = plsc.load_gather(hn_v, [go_vec])
        en_o = plsc.load_gather(en_v, [go_vec])
        wn2 = jnp.maximum(dot_ww, 1e-30)
        wn = wn2 * _rsqrt_newton(wn2)
        sim_h = dot_hw / jnp.maximum(hn_o * wn, 1e-8)
        sim_e = dot_ew / jnp.maximum(en_o * wn, 1e-8)
        gap = sim_h - sim_e
        return s_abs + jnp.abs(gap), s_loss + _sigmoid_sq(gap)

    a_abs, a_loss = lax.fori_loop(0, NSS, anc_step, (zeros, zeros))

    psum_v[0, :] = g_abs
    psum_v[1, :] = g_loss
    psum_v[2, :] = a_abs
    psum_v[3, :] = a_loss
    pltpu.sync_copy(psum_v, out_hbm.at[wid])


def _sc_pair_loss(h, e, w, hn, en):
    mesh = plsc.VectorSubcoreMesh(core_axis_name="c", subcore_axis_name="s")
    f = pl.kernel(
        _sc_body,
        mesh=mesh,
        compiler_params=pltpu.CompilerParams(needs_layout_passes=False,
                                             skip_device_barrier=True),
        out_type=jax.ShapeDtypeStruct((NW, 4, LANES), jnp.float32),
        scratch_types=[
            pltpu.VMEM((BATCH,), jnp.float32),
            pltpu.VMEM((BATCH,), jnp.float32),
            pltpu.VMEM((BATCH,), jnp.int32),
            pltpu.VMEM((PT,), jnp.int32),
            pltpu.VMEM((PT,), jnp.int32),
            pltpu.VMEM((PT,), jnp.int32),
            pltpu.VMEM((PT,), jnp.int32),
            pltpu.VMEM((PT,), jnp.uint32),
            pltpu.VMEM((2 * PT,), jnp.int32),
            pltpu.VMEM((2 * PT,), jnp.int32),
            pltpu.VMEM((2 * PT,), jnp.int32),
            pltpu.VMEM((2 * PT,), jnp.int32),
            pltpu.VMEM((PT,), jnp.uint32),
            pltpu.VMEM((PT,), jnp.uint32),
            pltpu.VMEM((PT,), jnp.uint32),
            pltpu.VMEM((PT,), jnp.uint32),
            pltpu.VMEM((PT,), jnp.uint32),
            pltpu.VMEM((PT,), jnp.uint32),
        ] + [pltpu.VMEM((CH, DIM), jnp.float32)] * 16 + [
            pltpu.VMEM((4, LANES), jnp.float32),
            pltpu.SemaphoreType.DMA,
            pltpu.SemaphoreType.DMA,
            pltpu.SemaphoreType.DMA,
            pltpu.SemaphoreType.DMA,
        ],
    )
    return f(h, e, w, hn, en, jnp.asarray(_CONSTS))


def kernel(hidden_states, target_embeddings, embedding_weight):
    hn, en = _row_norms(hidden_states, target_embeddings)
    partials = _sc_pair_loss(hidden_states, target_embeddings,
                             embedding_weight, hn, en)
    sums = jnp.sum(partials, axis=(0, 2))
    inv = jnp.float32(1.0 / NUM_PAIRS)
    geo_gap = sums[0] * inv
    geo_loss = sums[1] * inv
    anc_gap = sums[2] * inv
    anc_loss = sums[3] * inv
    total = geo_loss + 0.5 * anc_loss
    raw_gap = geo_gap + 0.5 * anc_gap
    return (total, geo_loss, anc_loss, raw_gap)


# speedup vs baseline: 1.2962x; 1.0186x over previous
import jax
import jax.numpy as jnp
import numpy as np
from jax import lax
from jax.experimental import pallas as pl
from jax.experimental.pallas import tpu as pltpu
from jax.experimental.pallas import tpu_sc as plsc

BATCH = 8192
DIM = 1024
VOCAB = 100000
NUM_PAIRS = BATCH // 2
NC = 2
NS = 16
LANES = 16
NW = NC * NS
PT = NUM_PAIRS // NW
CH = 4
NCH = PT // CH
NSS = NCH // 4
NG = PT // LANES
NK = DIM // LANES
NMASK = BATCH // LANES
SIGMOID_SCALE = 10.0


_U32 = np.uint32


def _tf2x32(k1, k2, c1, c2):
    rot0 = (13, 15, 26, 6)
    rot1 = (17, 29, 16, 24)
    ks0 = _U32(k1)
    ks1 = _U32(k2)
    ks2 = _U32(ks0 ^ ks1 ^ _U32(0x1BD11BDA))
    x0 = c1.astype(_U32)
    x1 = c2.astype(_U32)
    with np.errstate(over="ignore"):
        x0 = x0 + ks0
        x1 = x1 + ks1

        def rounds(x0, x1, rots):
            for r in rots:
                x0 = (x0 + x1).astype(_U32)
                x1 = ((x1 << _U32(r)) | (x1 >> _U32(32 - r))).astype(_U32)
                x1 = x1 ^ x0
            return x0, x1

        for i, rots in enumerate((rot0, rot1, rot0, rot1, rot0)):
            x0, x1 = rounds(x0, x1, rots)
            ka, kb = ((ks1, ks2), (ks2, ks0), (ks0, ks1),
                      (ks1, ks2), (ks2, ks0))[i]
            x0 = (x0 + ka).astype(_U32)
            x1 = (x1 + kb + _U32(i + 1)).astype(_U32)
    return x0, x1


def _np_split(kpair, num):
    b1, b2 = _tf2x32(kpair[0], kpair[1], np.zeros(num, _U32),
                     np.arange(num, dtype=_U32))
    return np.stack([b1, b2], axis=1)


def _np_bits(kpair, size):
    b1, b2 = _tf2x32(kpair[0], kpair[1], np.zeros(size, _U32),
                     np.arange(size, dtype=_U32))
    return b1 ^ b2


def _pair_constants():
    kd = np.array([0, 42], _U32)
    ki, kj, ko, ke = _np_split(kd, 4)

    def bits2(kp):
        kk = _np_split(kp, 2)
        return _np_bits(kk[0], NUM_PAIRS), _np_bits(kk[1], NUM_PAIRS)

    hbi, lbi = bits2(ki)
    hbj, lbj = bits2(kj)
    hbo, lbo = bits2(ko)
    hbe, lbe = bits2(ke)
    span = _U32(VOCAB)
    with np.errstate(over="ignore"):
        mult = _U32(65536) % span
        mult = _U32(mult * mult) % span
        emb = (((hbe % span) * mult + (lbe % span)) % span).astype(np.int32)
    return np.concatenate([hbi, lbi, hbj, lbj, hbo, lbo,
                           emb.view(np.uint32)])


_CONSTS = _pair_constants()


def _norm_body(h_ref, e_ref, hn_ref, en_ref):
    h = h_ref[...]
    e = e_ref[...]
    hn_ref[...] = jnp.sqrt(jnp.sum(h * h, axis=-1))
    en_ref[...] = jnp.sqrt(jnp.sum(e * e, axis=-1))


def _row_norms(hidden, target):
    h3 = hidden.reshape(64, 128, DIM)
    e3 = target.reshape(64, 128, DIM)
    hn, en = pl.pallas_call(
        _norm_body,
        grid=(8,),
        in_specs=[
            pl.BlockSpec((8, 128, DIM), lambda i: (i, 0, 0)),
            pl.BlockSpec((8, 128, DIM), lambda i: (i, 0, 0)),
        ],
        out_specs=[
            pl.BlockSpec((8, 128), lambda i: (i, 0)),
            pl.BlockSpec((8, 128), lambda i: (i, 0)),
        ],
        out_shape=[
            jax.ShapeDtypeStruct((64, 128), jnp.float32),
            jax.ShapeDtypeStruct((64, 128), jnp.float32),
        ],
    )(h3, e3)
    return hn.reshape(BATCH), en.reshape(BATCH)


_GATHER_DNUMS = lax.GatherDimensionNumbers(
    offset_dims=(), collapsed_slice_dims=(0,), start_index_map=(0,))


def _lane_perm(v, idx):
    return lax.gather(v, idx.reshape(LANES, 1), _GATHER_DNUMS, (1,),
                      mode=lax.GatherScatterMode.PROMISE_IN_BOUNDS)


def _lane_allsum(v):
    lanes = lax.iota(jnp.int32, LANES)
    for k in (1, 2, 4, 8):
        v = v + _lane_perm(v, lanes ^ k)
    return v


def _rsqrt_newton(x):
    i = plsc.bitcast(x, jnp.int32)
    i = jnp.int32(0x5F3759DF) - lax.shift_right_arithmetic(i, 1)
    y = plsc.bitcast(i, jnp.float32)
    for _ in range(3):
        y = y * (1.5 - 0.5 * x * y * y)
    return y


def _sigmoid_sq(gap):
    s = 1.0 / (1.0 + jnp.exp(-SIGMOID_SCALE * gap))
    d = s - 0.5
    return d * d


def _randint_vec(hb, lb, span):
    mult = jnp.uint32(65536) % span
    mult = (mult * mult) % span
    off = ((hb % span) * mult + (lb % span)) % span
    return off.astype(jnp.int32)


def _gather_start(table, idx_slice, buf, sem):
    pltpu.make_async_copy(table.at[idx_slice], buf, sem).start()


def _gather_wait(table, idx_slice, buf, sem):
    pltpu.make_async_copy(table.at[idx_slice], buf, sem).wait()


def _sc_body(h_hbm, e_hbm, w_hbm, hn_hbm, en_hbm, cb_hbm,
             out_hbm,
             hn_v, en_v, vi_v, gi_v, gj_v, go_v, we_v, weu_v,
             gi_p, gj_p, go_p, we_p,
             hbi_v, lbi_v, hbj_v, lbj_v, hbo_v, lbo_v,
             ba0, ba1, ba2, ba3, bb0, bb1, bb2, bb3,
             bc0, bc1, bc2, bc3, bd0, bd1, bd2, bd3, psum_v,
             sm0, sm1, sm2, sm3):
    wid = lax.axis_index("s") * NC + lax.axis_index("c")
    base = wid * PT
    pltpu.sync_copy(hn_hbm, hn_v)
    pltpu.sync_copy(en_hbm, en_v)
    pltpu.sync_copy(cb_hbm.at[pl.ds(0 * NUM_PAIRS + base, PT)], hbi_v)
    pltpu.sync_copy(cb_hbm.at[pl.ds(1 * NUM_PAIRS + base, PT)], lbi_v)
    pltpu.sync_copy(cb_hbm.at[pl.ds(2 * NUM_PAIRS + base, PT)], hbj_v)
    pltpu.sync_copy(cb_hbm.at[pl.ds(3 * NUM_PAIRS + base, PT)], lbj_v)
    pltpu.sync_copy(cb_hbm.at[pl.ds(4 * NUM_PAIRS + base, PT)], hbo_v)
    pltpu.sync_copy(cb_hbm.at[pl.ds(5 * NUM_PAIRS + base, PT)], lbo_v)
    pltpu.sync_copy(cb_hbm.at[pl.ds(6 * NUM_PAIRS + base, PT)], weu_v)

    zeros = jnp.zeros((LANES,), jnp.float32)
    lanes = lax.iota(jnp.int32, LANES)
    last = jnp.full((LANES,), LANES - 1, jnp.int32)

    vi_v[pl.ds(0, LANES)] = jnp.zeros((LANES,), jnp.int32)

    def mask_body(i, off):
        sl = pl.ds(i * LANES, LANES)
        m = (hn_v[sl] > 1e-8) & (en_v[sl] > 1e-8)
        mi = m.astype(jnp.int32)
        cs = plsc.cumsum(mi)
        plsc.store_scatter(vi_v, [off + cs - 1], lanes + i * LANES, mask=m)
        return off + plsc.all_reduce_population_count(m)

    vb_vec = lax.fori_loop(0, NMASK, mask_body,
                           jnp.zeros((LANES,), jnp.int32))

    span_i = jnp.maximum(vb_vec, 1).astype(jnp.uint32)
    span_j = jnp.maximum(vb_vec - 1, 1).astype(jnp.uint32)

    def gen_body(g, carry):
        sl = pl.ds(g * LANES, LANES)
        ii = _randint_vec(hbi_v[sl], lbi_v[sl], span_i)
        jj = _randint_vec(hbj_v[sl], lbj_v[sl], span_j)
        jj = jj + (jj >= ii).astype(jnp.int32)
        oo = _randint_vec(hbo_v[sl], lbo_v[sl], span_i)
        giv = plsc.load_gather(vi_v, [ii])
        gjv = plsc.load_gather(vi_v, [jj])
        gov = plsc.load_gather(vi_v, [oo])
        wev = plsc.bitcast(weu_v[sl], jnp.int32)
        gi_v[sl] = giv
        gj_v[sl] = gjv
        go_v[sl] = gov
        we_v[sl] = wev
        pos = (lanes & 3) + ((lanes >> 2) * 8) + g * 32
        plsc.store_scatter(gi_p, [pos], giv)
        plsc.store_scatter(gj_p, [pos], gjv)
        plsc.store_scatter(go_p, [pos], gov)
        plsc.store_scatter(we_p, [pos], wev)
        return carry

    lax.fori_loop(0, NG, gen_body, 0)

    def idx4(ref, c):
        return ref.at[pl.ds(c * 8, CH)]

    bufs_a = (ba0, ba1, ba2, ba3)
    bufs_b = (bb0, bb1, bb2, bb3)
    bufs_c = (bc0, bc1, bc2, bc3)
    bufs_d = (bd0, bd1, bd2, bd3)
    sems = (sm0, sm1, sm2, sm3)

    def anc_issue(c, j):
        _gather_start(h_hbm, idx4(go_p, c), bufs_a[j], sems[j])
        _gather_start(w_hbm, idx4(we_p, c), bufs_b[j], sems[j])
        _gather_start(e_hbm, idx4(go_p, c), bufs_c[j], sems[j])

    def anc_wait(c, j):
        _gather_wait(h_hbm, idx4(go_p, c), bufs_a[j], sems[j])
        _gather_wait(w_hbm, idx4(we_p, c), bufs_b[j], sems[j])
        _gather_wait(e_hbm, idx4(go_p, c), bufs_c[j], sems[j])

    def geo_issue(c, j):
        _gather_start(h_hbm, idx4(gi_p, c), bufs_a[j], sems[j])
        _gather_start(h_hbm, idx4(gj_p, c), bufs_b[j], sems[j])
        _gather_start(e_hbm, idx4(gi_p, c), bufs_c[j], sems[j])
        _gather_start(e_hbm, idx4(gj_p, c), bufs_d[j], sems[j])

    def geo_wait(c, j):
        _gather_wait(h_hbm, idx4(gi_p, c), bufs_a[j], sems[j])
        _gather_wait(h_hbm, idx4(gj_p, c), bufs_b[j], sems[j])
        _gather_wait(e_hbm, idx4(gi_p, c), bufs_c[j], sems[j])
        _gather_wait(e_hbm, idx4(gj_p, c), bufs_d[j], sems[j])

    for j in range(4):
        geo_issue(j, j)

    def geo_step(s, carry):
        s_abs, s_loss = carry
        dot_h = zeros
        dot_e = zeros
        for j in range(4):
            c = 4 * s + j
            geo_wait(c, j)
            ba, bb, bc, bd = bufs_a[j], bufs_b[j], bufs_c[j], bufs_d[j]
            for p in range(CH):
                def kbody(k, acc, _p=p, _ba=ba, _bb=bb, _bc=bc, _bd=bd):
                    ah, ae = acc
                    ko = k * LANES
                    ah = ah + _ba[_p, pl.ds(ko, LANES)] * _bb[_p, pl.ds(ko, LANES)]
                    ae = ae + _bc[_p, pl.ds(ko, LANES)] * _bd[_p, pl.ds(ko, LANES)]
                    return ah, ae
                ah, ae = lax.fori_loop(0, NK, kbody, (zeros, zeros), unroll=8)
                sel = lanes == (4 * j + p)
                dot_h = jnp.where(sel, _lane_allsum(ah), dot_h)
                dot_e = jnp.where(sel, _lane_allsum(ae), dot_e)

            @pl.when(s < NSS - 1)
            def _(c=c, j=j):
                geo_issue(c + 4, j)

            @pl.when(s == NSS - 1)
            def _(j=j):
                anc_issue(j, j)

        sl = pl.ds(s * LANES, LANES)
        gi_vec = gi_v[sl]
        gj_vec = gj_v[sl]
        hn_i = plsc.load_gather(hn_v, [gi_vec])
        hn_j = plsc.load_gather(hn_v, [gj_vec])
        en_i = plsc.load_gather(en_v, [gi_vec])
        en_j = plsc.load_gather(en_v, [gj_vec])
        sim_h = dot_h / jnp.maximum(hn_i * hn_j, 1e-8)
        sim_e = dot_e / jnp.maximum(en_i * en_j, 1e-8)
        gap = sim_h - sim_e
        return s_abs + jnp.abs(gap), s_loss + _sigmoid_sq(gap)

    g_abs, g_loss = lax.fori_loop(0, NSS, geo_step, (zeros, zeros))

    def anc_step(s, carry):
        s_abs, s_loss = carry
        dot_hw = zeros
        dot_ew = zeros
        dot_ww = zeros
        for j in range(4):
            c = 4 * s + j
            anc_wait(c, j)
            ba, bb, bc = bufs_a[j], bufs_b[j], bufs_c[j]
            for p in range(CH):
                def kbody(k, acc, _p=p, _ba=ba, _bb=bb, _bc=bc):
                    aw, ew, ww = acc
                    ko = k * LANES
                    wv = _bb[_p, pl.ds(ko, LANES)]
                    aw = aw + _ba[_p, pl.ds(ko, LANES)] * wv
                    ew = ew + _bc[_p, pl.ds(ko, LANES)] * wv
                    ww = ww + wv * wv
                    return aw, ew, ww
                aw, ew, ww = lax.fori_loop(0, NK, kbody,
                                           (zeros, zeros, zeros), unroll=8)
                sel = lanes == (4 * j + p)
                dot_hw = jnp.where(sel, _lane_allsum(aw), dot_hw)
                dot_ew = jnp.where(sel, _lane_allsum(ew), dot_ew)
                dot_ww = jnp.where(sel, _lane_allsum(ww), dot_ww)

            @pl.when(s < NSS - 1)
            def _(c=c, j=j):
                anc_issue(c + 4, j)

        sl = pl.ds(s * LANES, LANES)
        go_vec = go_v[sl]
        hn_o = plsc.load_gather(hn_v, [go_vec])
        en_o = plsc.load_gather(en_v, [go_vec])
        wn2 = jnp.maximum(dot_ww, 1e-30)
        wn = wn2 * _rsqrt_newton(wn2)
        sim_h = dot_hw / jnp.maximum(hn_o * wn, 1e-8)
        sim_e = dot_ew / jnp.maximum(en_o * wn, 1e-8)
        gap = sim_h - sim_e
        return s_abs + jnp.abs(gap), s_loss + _sigmoid_sq(gap)

    a_abs, a_loss = lax.fori_loop(0, NSS, anc_step, (zeros, zeros))

    psum_v[0, :] = g_abs
    psum_v[1, :] = g_loss
    psum_v[2, :] = a_abs
    psum_v[3, :] = a_loss
    pltpu.sync_copy(psum_v, out_hbm.at[wid])


def _sc_pair_loss(h, e, w, hn, en):
    mesh = plsc.VectorSubcoreMesh(core_axis_name="c", subcore_axis_name="s")
    f = pl.kernel(
        _sc_body,
        mesh=mesh,
        compiler_params=pltpu.CompilerParams(needs_layout_passes=False,
                                             skip_device_barrier=True),
        out_type=jax.ShapeDtypeStruct((NW, 4, LANES), jnp.float32),
        scratch_types=[
            pltpu.VMEM((BATCH,), jnp.float32),
            pltpu.VMEM((BATCH,), jnp.float32),
            pltpu.VMEM((BATCH,), jnp.int32),
            pltpu.VMEM((PT,), jnp.int32),
            pltpu.VMEM((PT,), jnp.int32),
            pltpu.VMEM((PT,), jnp.int32),
            pltpu.VMEM((PT,), jnp.int32),
            pltpu.VMEM((PT,), jnp.uint32),
            pltpu.VMEM((2 * PT,), jnp.int32),
            pltpu.VMEM((2 * PT,), jnp.int32),
            pltpu.VMEM((2 * PT,), jnp.int32),
            pltpu.VMEM((2 * PT,), jnp.int32),
            pltpu.VMEM((PT,), jnp.uint32),
            pltpu.VMEM((PT,), jnp.uint32),
            pltpu.VMEM((PT,), jnp.uint32),
            pltpu.VMEM((PT,), jnp.uint32),
            pltpu.VMEM((PT,), jnp.uint32),
            pltpu.VMEM((PT,), jnp.uint32),
        ] + [pltpu.VMEM((CH, DIM), jnp.float32)] * 16 + [
            pltpu.VMEM((4, LANES), jnp.float32),
            pltpu.SemaphoreType.DMA,
            pltpu.SemaphoreType.DMA,
            pltpu.SemaphoreType.DMA,
            pltpu.SemaphoreType.DMA,
        ],
    )
    return f(h, e, w, hn, en, jnp.asarray(_CONSTS))


def kernel(hidden_states, target_embeddings, embedding_weight):
    hn, en = _row_norms(hidden_states, target_embeddings)
    partials = _sc_pair_loss(hidden_states, target_embeddings,
                             embedding_weight, hn, en)
    sums = jnp.sum(partials, axis=(0, 2))
    inv = jnp.float32(1.0 / NUM_PAIRS)
    geo_gap = sums[0] * inv
    geo_loss = sums[1] * inv
    anc_gap = sums[2] * inv
    anc_loss = sums[3] * inv
    total = geo_loss + 0.5 * anc_loss
    raw_gap = geo_gap + 0.5 * anc_gap
    return (total, geo_loss, anc_loss, raw_gap)


# speedup vs baseline: 1.3030x; 1.0053x over previous
import jax
import jax.numpy as jnp
import numpy as np
from jax import lax
from jax.experimental import pallas as pl
from jax.experimental.pallas import tpu as pltpu
from jax.experimental.pallas import tpu_sc as plsc

BATCH = 8192
DIM = 1024
VOCAB = 100000
NUM_PAIRS = BATCH // 2
NC = 2
NS = 16
LANES = 16
NW = NC * NS
PT = NUM_PAIRS // NW
CH = 4
NCH = PT // CH
NSS = NCH // 4
NG = PT // LANES
NK = DIM // LANES
NMASK = BATCH // LANES
SIGMOID_SCALE = 10.0


_U32 = np.uint32


def _tf2x32(k1, k2, c1, c2):
    rot0 = (13, 15, 26, 6)
    rot1 = (17, 29, 16, 24)
    ks0 = _U32(k1)
    ks1 = _U32(k2)
    ks2 = _U32(ks0 ^ ks1 ^ _U32(0x1BD11BDA))
    x0 = c1.astype(_U32)
    x1 = c2.astype(_U32)
    with np.errstate(over="ignore"):
        x0 = x0 + ks0
        x1 = x1 + ks1

        def rounds(x0, x1, rots):
            for r in rots:
                x0 = (x0 + x1).astype(_U32)
                x1 = ((x1 << _U32(r)) | (x1 >> _U32(32 - r))).astype(_U32)
                x1 = x1 ^ x0
            return x0, x1

        for i, rots in enumerate((rot0, rot1, rot0, rot1, rot0)):
            x0, x1 = rounds(x0, x1, rots)
            ka, kb = ((ks1, ks2), (ks2, ks0), (ks0, ks1),
                      (ks1, ks2), (ks2, ks0))[i]
            x0 = (x0 + ka).astype(_U32)
            x1 = (x1 + kb + _U32(i + 1)).astype(_U32)
    return x0, x1


def _np_split(kpair, num):
    b1, b2 = _tf2x32(kpair[0], kpair[1], np.zeros(num, _U32),
                     np.arange(num, dtype=_U32))
    return np.stack([b1, b2], axis=1)


def _np_bits(kpair, size):
    b1, b2 = _tf2x32(kpair[0], kpair[1], np.zeros(size, _U32),
                     np.arange(size, dtype=_U32))
    return b1 ^ b2


def _pair_constants():
    kd = np.array([0, 42], _U32)
    ki, kj, ko, ke = _np_split(kd, 4)

    def bits2(kp):
        kk = _np_split(kp, 2)
        return _np_bits(kk[0], NUM_PAIRS), _np_bits(kk[1], NUM_PAIRS)

    hbi, lbi = bits2(ki)
    hbj, lbj = bits2(kj)
    hbo, lbo = bits2(ko)
    hbe, lbe = bits2(ke)
    span = _U32(VOCAB)
    with np.errstate(over="ignore"):
        mult = _U32(65536) % span
        mult = _U32(mult * mult) % span
        emb = (((hbe % span) * mult + (lbe % span)) % span).astype(np.int32)
    return np.concatenate([hbi, lbi, hbj, lbj, hbo, lbo,
                           emb.view(np.uint32)])


_CONSTS = _pair_constants()


def _norm_body(h_ref, e_ref, hn_ref, en_ref):
    h = h_ref[...]
    e = e_ref[...]
    hn_ref[...] = jnp.sqrt(jnp.sum(h * h, axis=-1))
    en_ref[...] = jnp.sqrt(jnp.sum(e * e, axis=-1))


def _row_norms(hidden, target):
    h3 = hidden.reshape(64, 128, DIM)
    e3 = target.reshape(64, 128, DIM)
    hn, en = pl.pallas_call(
        _norm_body,
        grid=(8,),
        in_specs=[
            pl.BlockSpec((8, 128, DIM), lambda i: (i, 0, 0)),
            pl.BlockSpec((8, 128, DIM), lambda i: (i, 0, 0)),
        ],
        out_specs=[
            pl.BlockSpec((8, 128), lambda i: (i, 0)),
            pl.BlockSpec((8, 128), lambda i: (i, 0)),
        ],
        out_shape=[
            jax.ShapeDtypeStruct((64, 128), jnp.float32),
            jax.ShapeDtypeStruct((64, 128), jnp.float32),
        ],
    )(h3, e3)
    return hn.reshape(BATCH), en.reshape(BATCH)


_GATHER_DNUMS = lax.GatherDimensionNumbers(
    offset_dims=(), collapsed_slice_dims=(0,), start_index_map=(0,))


def _lane_perm(v, idx):
    return lax.gather(v, idx.reshape(LANES, 1), _GATHER_DNUMS, (1,),
                      mode=lax.GatherScatterMode.PROMISE_IN_BOUNDS)


def _lane_allsum(v):
    lanes = lax.iota(jnp.int32, LANES)
    for k in (1, 2, 4, 8):
        v = v + _lane_perm(v, lanes ^ k)
    return v


def _rsqrt_newton(x):
    i = plsc.bitcast(x, jnp.int32)
    i = jnp.int32(0x5F3759DF) - lax.shift_right_arithmetic(i, 1)
    y = plsc.bitcast(i, jnp.float32)
    for _ in range(3):
        y = y * (1.5 - 0.5 * x * y * y)
    return y


def _sigmoid_sq(gap):
    s = 1.0 / (1.0 + jnp.exp(-SIGMOID_SCALE * gap))
    d = s - 0.5
    return d * d


def _randint_vec(hb, lb, span):
    mult = jnp.uint32(65536) % span
    mult = (mult * mult) % span
    off = ((hb % span) * mult + (lb % span)) % span
    return off.astype(jnp.int32)


def _gather_start(table, idx_slice, buf, sem):
    pltpu.make_async_copy(table.at[idx_slice], buf, sem).start()


def _gather_wait(table, idx_slice, buf, sem):
    pltpu.make_async_copy(table.at[idx_slice], buf, sem).wait()


def _sc_body(h_hbm, e_hbm, w_hbm, hn_hbm, en_hbm, cb_hbm,
             out_hbm,
             hn_v, en_v, vi_v, gi_v, gj_v, go_v, we_v, weu_v,
             gi_p, gj_p, go_p, we_p,
             hbi_v, lbi_v, hbj_v, lbj_v, hbo_v, lbo_v,
             ba0, ba1, ba2, ba3, bb0, bb1, bb2, bb3,
             bc0, bc1, bc2, bc3, bd0, bd1, bd2, bd3, psum_v,
             sm0, sm1, sm2, sm3):
    wid = lax.axis_index("s") * NC + lax.axis_index("c")
    base = wid * PT
    pltpu.sync_copy(hn_hbm, hn_v)
    pltpu.sync_copy(en_hbm, en_v)
    pltpu.sync_copy(cb_hbm.at[pl.ds(0 * NUM_PAIRS + base, PT)], hbi_v)
    pltpu.sync_copy(cb_hbm.at[pl.ds(1 * NUM_PAIRS + base, PT)], lbi_v)
    pltpu.sync_copy(cb_hbm.at[pl.ds(2 * NUM_PAIRS + base, PT)], hbj_v)
    pltpu.sync_copy(cb_hbm.at[pl.ds(3 * NUM_PAIRS + base, PT)], lbj_v)
    pltpu.sync_copy(cb_hbm.at[pl.ds(4 * NUM_PAIRS + base, PT)], hbo_v)
    pltpu.sync_copy(cb_hbm.at[pl.ds(5 * NUM_PAIRS + base, PT)], lbo_v)
    pltpu.sync_copy(cb_hbm.at[pl.ds(6 * NUM_PAIRS + base, PT)], weu_v)

    zeros = jnp.zeros((LANES,), jnp.float32)
    lanes = lax.iota(jnp.int32, LANES)
    last = jnp.full((LANES,), LANES - 1, jnp.int32)

    vi_v[pl.ds(0, LANES)] = jnp.zeros((LANES,), jnp.int32)

    def mask_body(i, off):
        sl = pl.ds(i * LANES, LANES)
        m = (hn_v[sl] > 1e-8) & (en_v[sl] > 1e-8)
        mi = m.astype(jnp.int32)
        cs = plsc.cumsum(mi)
        plsc.store_scatter(vi_v, [off + cs - 1], lanes + i * LANES, mask=m)
        return off + plsc.all_reduce_population_count(m)

    vb_vec = lax.fori_loop(0, NMASK, mask_body,
                           jnp.zeros((LANES,), jnp.int32))

    span_i = jnp.maximum(vb_vec, 1).astype(jnp.uint32)
    span_j = jnp.maximum(vb_vec - 1, 1).astype(jnp.uint32)

    def gen_body(g, carry):
        sl = pl.ds(g * LANES, LANES)
        ii = _randint_vec(hbi_v[sl], lbi_v[sl], span_i)
        jj = _randint_vec(hbj_v[sl], lbj_v[sl], span_j)
        jj = jj + (jj >= ii).astype(jnp.int32)
        oo = _randint_vec(hbo_v[sl], lbo_v[sl], span_i)
        giv = plsc.load_gather(vi_v, [ii])
        gjv = plsc.load_gather(vi_v, [jj])
        gov = plsc.load_gather(vi_v, [oo])
        wev = plsc.bitcast(weu_v[sl], jnp.int32)
        gi_v[sl] = giv
        gj_v[sl] = gjv
        go_v[sl] = gov
        we_v[sl] = wev
        pos = (lanes & 3) + ((lanes >> 2) * 8) + g * 32
        plsc.store_scatter(gi_p, [pos], giv)
        plsc.store_scatter(gj_p, [pos], gjv)
        plsc.store_scatter(go_p, [pos], gov)
        plsc.store_scatter(we_p, [pos], wev)
        return carry

    lax.fori_loop(0, NG, gen_body, 0)

    def idx4(ref, c):
        return ref.at[pl.ds(c * 8, CH)]

    bufs_a = (ba0, ba1, ba2, ba3)
    bufs_b = (bb0, bb1, bb2, bb3)
    bufs_c = (bc0, bc1, bc2, bc3)
    bufs_d = (bd0, bd1, bd2, bd3)
    sems = (sm0, sm1, sm2, sm3)

    def anc_issue(c, j):
        _gather_start(h_hbm, idx4(go_p, c), bufs_a[j], sems[j])
        _gather_start(w_hbm, idx4(we_p, c), bufs_b[j], sems[j])
        _gather_start(e_hbm, idx4(go_p, c), bufs_c[j], sems[j])

    def anc_wait(c, j):
        _gather_wait(h_hbm, idx4(go_p, c), bufs_a[j], sems[j])
        _gather_wait(w_hbm, idx4(we_p, c), bufs_b[j], sems[j])
        _gather_wait(e_hbm, idx4(go_p, c), bufs_c[j], sems[j])

    def geo_issue(c, j):
        _gather_start(h_hbm, idx4(gi_p, c), bufs_a[j], sems[j])
        _gather_start(h_hbm, idx4(gj_p, c), bufs_b[j], sems[j])
        _gather_start(e_hbm, idx4(gi_p, c), bufs_c[j], sems[j])
        _gather_start(e_hbm, idx4(gj_p, c), bufs_d[j], sems[j])

    def geo_wait(c, j):
        _gather_wait(h_hbm, idx4(gi_p, c), bufs_a[j], sems[j])
        _gather_wait(h_hbm, idx4(gj_p, c), bufs_b[j], sems[j])
        _gather_wait(e_hbm, idx4(gi_p, c), bufs_c[j], sems[j])
        _gather_wait(e_hbm, idx4(gj_p, c), bufs_d[j], sems[j])

    for j in range(4):
        geo_issue(j, j)

    def geo_step(s, carry):
        s_abs, s_loss = carry
        dot_h = zeros
        dot_e = zeros
        for j in range(4):
            c = 4 * s + j
            geo_wait(c, j)
            ba, bb, bc, bd = bufs_a[j], bufs_b[j], bufs_c[j], bufs_d[j]
            for p in range(CH):
                def kbody(k, acc, _p=p, _ba=ba, _bb=bb, _bc=bc, _bd=bd):
                    ah, ae = acc
                    ko = k * LANES
                    ah = ah + _ba[_p, pl.ds(ko, LANES)] * _bb[_p, pl.ds(ko, LANES)]
                    ae = ae + _bc[_p, pl.ds(ko, LANES)] * _bd[_p, pl.ds(ko, LANES)]
                    return ah, ae
                ah, ae = lax.fori_loop(0, NK, kbody, (zeros, zeros), unroll=4)
                sel = lanes == (4 * j + p)
                dot_h = jnp.where(sel, _lane_allsum(ah), dot_h)
                dot_e = jnp.where(sel, _lane_allsum(ae), dot_e)

            @pl.when(s < NSS - 1)
            def _(c=c, j=j):
                geo_issue(c + 4, j)

            @pl.when(s == NSS - 1)
            def _(j=j):
                anc_issue(j, j)

        sl = pl.ds(s * LANES, LANES)
        gi_vec = gi_v[sl]
        gj_vec = gj_v[sl]
        hn_i = plsc.load_gather(hn_v, [gi_vec])
        hn_j = plsc.load_gather(hn_v, [gj_vec])
        en_i = plsc.load_gather(en_v, [gi_vec])
        en_j = plsc.load_gather(en_v, [gj_vec])
        sim_h = dot_h / jnp.maximum(hn_i * hn_j, 1e-8)
        sim_e = dot_e / jnp.maximum(en_i * en_j, 1e-8)
        gap = sim_h - sim_e
        return s_abs + jnp.abs(gap), s_loss + _sigmoid_sq(gap)

    g_abs, g_loss = lax.fori_loop(0, NSS, geo_step, (zeros, zeros))

    def anc_step(s, carry):
        s_abs, s_loss = carry
        dot_hw = zeros
        dot_ew = zeros
        dot_ww = zeros
        for j in range(4):
            c = 4 * s + j
            anc_wait(c, j)
            ba, bb, bc = bufs_a[j], bufs_b[j], bufs_c[j]
            for p in range(CH):
                def kbody(k, acc, _p=p, _ba=ba, _bb=bb, _bc=bc):
                    aw, ew, ww = acc
                    ko = k * LANES
                    wv = _bb[_p, pl.ds(ko, LANES)]
                    aw = aw + _ba[_p, pl.ds(ko, LANES)] * wv
                    ew = ew + _bc[_p, pl.ds(ko, LANES)] * wv
                    ww = ww + wv * wv
                    return aw, ew, ww
                aw, ew, ww = lax.fori_loop(0, NK, kbody,
                                           (zeros, zeros, zeros), unroll=4)
                sel = lanes == (4 * j + p)
                dot_hw = jnp.where(sel, _lane_allsum(aw), dot_hw)
                dot_ew = jnp.where(sel, _lane_allsum(ew), dot_ew)
                dot_ww = jnp.where(sel, _lane_allsum(ww), dot_ww)

            @pl.when(s < NSS - 1)
            def _(c=c, j=j):
                anc_issue(c + 4, j)

        sl = pl.ds(s * LANES, LANES)
        go_vec = go_v[sl]
        hn_o = plsc.load_gather(hn_v, [go_vec])
        en_o = plsc.load_gather(en_v, [go_vec])
        wn2 = jnp.maximum(dot_ww, 1e-30)
        wn = wn2 * _rsqrt_newton(wn2)
        sim_h = dot_hw / jnp.maximum(hn_o * wn, 1e-8)
        sim_e = dot_ew / jnp.maximum(en_o * wn, 1e-8)
        gap = sim_h - sim_e
        return s_abs + jnp.abs(gap), s_loss + _sigmoid_sq(gap)

    a_abs, a_loss = lax.fori_loop(0, NSS, anc_step, (zeros, zeros))

    psum_v[0, :] = g_abs
    psum_v[1, :] = g_loss
    psum_v[2, :] = a_abs
    psum_v[3, :] = a_loss
    pltpu.sync_copy(psum_v, out_hbm.at[wid])


def _sc_pair_loss(h, e, w, hn, en):
    mesh = plsc.VectorSubcoreMesh(core_axis_name="c", subcore_axis_name="s")
    f = pl.kernel(
        _sc_body,
        mesh=mesh,
        compiler_params=pltpu.CompilerParams(needs_layout_passes=False,
                                             skip_device_barrier=True),
        out_type=jax.ShapeDtypeStruct((NW, 4, LANES), jnp.float32),
        scratch_types=[
            pltpu.VMEM((BATCH,), jnp.float32),
            pltpu.VMEM((BATCH,), jnp.float32),
            pltpu.VMEM((BATCH,), jnp.int32),
            pltpu.VMEM((PT,), jnp.int32),
            pltpu.VMEM((PT,), jnp.int32),
            pltpu.VMEM((PT,), jnp.int32),
            pltpu.VMEM((PT,), jnp.int32),
            pltpu.VMEM((PT,), jnp.uint32),
            pltpu.VMEM((2 * PT,), jnp.int32),
            pltpu.VMEM((2 * PT,), jnp.int32),
            pltpu.VMEM((2 * PT,), jnp.int32),
            pltpu.VMEM((2 * PT,), jnp.int32),
            pltpu.VMEM((PT,), jnp.uint32),
            pltpu.VMEM((PT,), jnp.uint32),
            pltpu.VMEM((PT,), jnp.uint32),
            pltpu.VMEM((PT,), jnp.uint32),
            pltpu.VMEM((PT,), jnp.uint32),
            pltpu.VMEM((PT,), jnp.uint32),
        ] + [pltpu.VMEM((CH, DIM), jnp.float32)] * 16 + [
            pltpu.VMEM((4, LANES), jnp.float32),
            pltpu.SemaphoreType.DMA,
            pltpu.SemaphoreType.DMA,
            pltpu.SemaphoreType.DMA,
            pltpu.SemaphoreType.DMA,
        ],
    )
    return f(h, e, w, hn, en, jnp.asarray(_CONSTS))


def kernel(hidden_states, target_embeddings, embedding_weight):
    hn, en = _row_norms(hidden_states, target_embeddings)
    partials = _sc_pair_loss(hidden_states, target_embeddings,
                             embedding_weight, hn, en)
    sums = jnp.sum(partials, axis=(0, 2))
    inv = jnp.float32(1.0 / NUM_PAIRS)
    geo_gap = sums[0] * inv
    geo_loss = sums[1] * inv
    anc_gap = sums[2] * inv
    anc_loss = sums[3] * inv
    total = geo_loss + 0.5 * anc_loss
    raw_gap = geo_gap + 0.5 * anc_gap
    return (total, geo_loss, anc_loss, raw_gap)
